# Initial kernel scaffold; baseline (speedup 1.0000x reference)
import jax
import jax.numpy as jnp
from jax.experimental import pallas as pl


def kernel(x, pos, edge_index, edge_attr, W1, b1, W2, b2, Wr1_0, br1_0, Wr2_0, br2_0, ws_0, Wr1_1, br1_1, Wr2_1, br2_1, ws_1):
    raise NotImplementedError("write your pallas kernel here")


# speedup vs baseline: 62.3540x; 62.3540x over previous
import numpy as np

import jax
import jax.numpy as jnp
from jax import lax
from jax.experimental import pallas as pl
from jax.experimental.pallas import tpu as pltpu
from jax.experimental.pallas import tpu_sc as plsc

N = 100000
E = 3200000
D_IN = 67
H = 32

LAN = 128
NC = 2
NS = 16
NW = NC * NS
EPW = 100352
EROWS = 25088
EPAD = NW * EPW
CHW = 2048
NCHUNK = EPW // CHW
NGRP = CHW // 16
NT = 100352
NTS = NT // NS

_F32 = jnp.float32
_I32 = jnp.int32
_f = np.float32
_MAGIC = np.int32(0x5F3759DF)


def _newton_r(dx, dy, dz):
    rsq = dx * dx + dy * dy + dz * dz
    bits = plsc.bitcast(rsq, _I32)
    y = plsc.bitcast(_MAGIC - lax.shift_right_logical(bits, 1), _F32)
    hr = rsq * _f(0.5)
    for _ in range(3):
        y = y * (_f(1.5) - hr * y * y)
    return rsq * y


def _k1_body(x_ref, w1_ref, b1_ref, w2_ref, b2_ref, o_ref):
    xb = x_ref[...].astype(jnp.bfloat16)
    w1 = w1_ref[...].astype(jnp.bfloat16)
    l0 = jnp.dot(xb, w1, preferred_element_type=_F32)
    l0 = l0 + b1_ref[...]
    l0 = jnp.where(l0 > 0, l0, jnp.exp(l0) - _f(1.0))
    h = jnp.dot(l0.astype(jnp.bfloat16), w2_ref[...].astype(jnp.bfloat16),
                preferred_element_type=_F32) + b2_ref[0, 0]
    o_ref[...] = h


def _node_mlp(x, W1, b1, W2, b2):
    nb = 1000
    return pl.pallas_call(
        _k1_body,
        grid=(N // nb,),
        in_specs=[
            pl.BlockSpec((nb, D_IN), lambda i: (i, 0)),
            pl.BlockSpec((D_IN, D_IN), lambda i: (0, 0)),
            pl.BlockSpec((1, D_IN), lambda i: (0, 0)),
            pl.BlockSpec((D_IN, 1), lambda i: (0, 0)),
            pl.BlockSpec(memory_space=pltpu.SMEM),
        ],
        out_specs=pl.BlockSpec((nb, 1), lambda i: (i, 0)),
        out_shape=jax.ShapeDtypeStruct((N, 1), _F32),
    )(x, W1, b1.reshape(1, D_IN), W2, b2.reshape(1, 1))


def _k2_body(src_h, dst_h, px_h, py_h, pz_h, r_h,
             px_s, py_s, pz_s,
             srcb, dstb, gxs, gys, gzs, gxd, gyd, gzd, rb):
    c = lax.axis_index("c")
    s = lax.axis_index("s")
    w = s * NC + c
    sl = pl.ds(s * NTS, NTS)
    pltpu.sync_copy(px_h.at[sl], px_s.at[sl])
    pltpu.sync_copy(py_h.at[sl], py_s.at[sl])
    pltpu.sync_copy(pz_h.at[sl], pz_s.at[sl])
    plsc.subcore_barrier()

    def chunk(i, _):
        base = w * EPW + i * CHW
        ds = pl.ds(base, CHW)
        pltpu.sync_copy(src_h.at[ds], srcb)
        pltpu.sync_copy(dst_h.at[ds], dstb)
        pltpu.sync_copy(px_s.at[srcb], gxs)
        pltpu.sync_copy(py_s.at[srcb], gys)
        pltpu.sync_copy(pz_s.at[srcb], gzs)
        pltpu.sync_copy(px_s.at[dstb], gxd)
        pltpu.sync_copy(py_s.at[dstb], gyd)
        pltpu.sync_copy(pz_s.at[dstb], gzd)

        def grp(g, _):
            q = pl.ds(g * 16, 16)
            dx = gxs[q] - gxd[q]
            dy = gys[q] - gyd[q]
            dz = gzs[q] - gzd[q]
            rb[q] = _newton_r(dx, dy, dz)
            return 0
        lax.fori_loop(0, NGRP, grp, 0)
        pltpu.sync_copy(rb, r_h.at[ds])
        return 0

    lax.fori_loop(0, NCHUNK, chunk, 0)


def _edge_r(src1, dst1, posx, posy, posz):
    f = pl.kernel(
        _k2_body,
        out_type=jax.ShapeDtypeStruct((EPAD,), _F32),
        mesh=plsc.VectorSubcoreMesh(core_axis_name="c", subcore_axis_name="s"),
        compiler_params=pltpu.CompilerParams(needs_layout_passes=False),
        scratch_types=(
            [pltpu.VMEM_SHARED((NT,), _F32)] * 3
            + [pltpu.VMEM((CHW,), _I32)] * 2
            + [pltpu.VMEM((CHW,), _F32)] * 7
        ),
    )
    return f(src1, dst1, posx, posy, posz)


def _k3_body(e0_ref, e1_ref, r_ref, w10, b10, w20, b20, w11, b11, w21, b21,
             a0_ref, a1_ref, b1_ref):
    def b16(v):
        return v.astype(jnp.bfloat16).astype(_F32)

    e0 = b16(e0_ref[...])
    e1 = b16(e1_ref[...])
    rr = b16(r_ref[...])
    acc0 = jnp.zeros(e0.shape, _F32)
    acc1 = jnp.zeros(e0.shape, _F32)
    accb = jnp.zeros(e0.shape, _F32)
    for j in range(H):
        h0 = e0 * b16(w10[0, j]) + e1 * b16(w10[1, j]) + rr * b16(w10[2, j])
        h0 = jnp.maximum(h0 + b10[0, j], _f(0.0))
        h0 = b16(h0)
        acc0 = acc0 + h0 * b16(w20[j, 0])
        h1 = e0 * b16(w11[0, j]) + e1 * b16(w11[1, j]) + rr * b16(w11[2, j])
        h1 = jnp.maximum(h1 + b11[0, j], _f(0.0))
        h1 = b16(h1)
        acc1 = acc1 + h1 * b16(w21[j, 0])
        accb = accb + h1 * b16(w21[j, 1])
    a0_ref[...] = acc0 + b20[0, 0]
    a1_ref[...] = acc1 + b21[0, 0]
    b1_ref[...] = accb + b21[0, 1]


def _radial(ea0, ea1, r2, Wr1_0, br1_0, Wr2_0, br2_0, Wr1_1, br1_1, Wr2_1, br2_1):
    rb = 512
    smem = pl.BlockSpec(memory_space=pltpu.SMEM)
    blk = pl.BlockSpec((rb, LAN), lambda i: (i, 0))
    return pl.pallas_call(
        _k3_body,
        grid=(EROWS // rb,),
        in_specs=[blk, blk, blk] + [smem] * 8,
        out_specs=[blk, blk, blk],
        out_shape=[jax.ShapeDtypeStruct((EROWS, LAN), _F32)] * 3,
    )(ea0, ea1, r2,
      Wr1_0, br1_0.reshape(1, H), Wr2_0, br2_0.reshape(1, 2),
      Wr1_1, br1_1.reshape(1, H), Wr2_1, br2_1.reshape(1, 2))


def _k4_body(src_h, dst_h, a0_h, h0_h, out_h,
             h0_s, agg_s,
             srcb, dstb, a0b, hsb, mb, zb):
    c = lax.axis_index("c")
    s = lax.axis_index("s")
    w = s * NC + c
    sl = pl.ds(s * NTS, NTS)
    pltpu.sync_copy(h0_h.at[sl], h0_s.at[sl])

    def zloop(i, _):
        zb[pl.ds(i * 16, 16)] = jnp.zeros((16,), _F32)
        return 0
    lax.fori_loop(0, NTS // 16, zloop, 0)
    pltpu.sync_copy(zb, agg_s.at[sl])
    plsc.subcore_barrier()

    def chunk(i, _):
        base = w * EPW + i * CHW
        ds = pl.ds(base, CHW)
        pltpu.sync_copy(src_h.at[ds], srcb)
        pltpu.sync_copy(dst_h.at[ds], dstb)
        pltpu.sync_copy(a0_h.at[ds], a0b)
        pltpu.sync_copy(h0_s.at[srcb], hsb)

        def grp(g, _):
            q = pl.ds(g * 16, 16)
            mb[q] = a0b[q] * hsb[q]
            return 0
        lax.fori_loop(0, NGRP, grp, 0)
        pltpu.sync_copy(mb, agg_s.at[dstb], add=True)
        return 0

    lax.fori_loop(0, NCHUNK, chunk, 0)
    plsc.subcore_barrier()
    pltpu.sync_copy(agg_s.at[sl], out_h.at[c, sl])


def _layer0(src1, dst1, a0f, h0p):
    f = pl.kernel(
        _k4_body,
        out_type=jax.ShapeDtypeStruct((NC, NT), _F32),
        mesh=plsc.VectorSubcoreMesh(core_axis_name="c", subcore_axis_name="s"),
        compiler_params=pltpu.CompilerParams(needs_layout_passes=False),
        scratch_types=[
            pltpu.VMEM_SHARED((NT,), _F32),
            pltpu.VMEM_SHARED((NT,), _F32),
            pltpu.VMEM((CHW,), _I32),
            pltpu.VMEM((CHW,), _I32),
            pltpu.VMEM((CHW,), _F32),
            pltpu.VMEM((CHW,), _F32),
            pltpu.VMEM((CHW,), _F32),
            pltpu.VMEM((NTS,), _F32),
        ],
    )
    return f(src1, dst1, a0f, h0p)


def _k6_body(src_h, dst_h, a1_h, b1_h, px_h, py_h, pz_h, h0_h, part_h, ws0_h,
             h0a_out, out_h,
             px_s, py_s, pz_s, h0a_s, ag0, ag1, ag2, ag3,
             srcb, dstb, a1b, b1b, gxs, gys, gzs, gxd, gyd, gzd, hsb,
             m0b, m1b, m2b, m3b,
             p0b, p1b, h0b, hab, wsb):
    c = lax.axis_index("c")
    s = lax.axis_index("s")
    w = s * NC + c
    sl = pl.ds(s * NTS, NTS)
    pltpu.sync_copy(px_h.at[sl], px_s.at[sl])
    pltpu.sync_copy(py_h.at[sl], py_s.at[sl])
    pltpu.sync_copy(pz_h.at[sl], pz_s.at[sl])
    pltpu.sync_copy(part_h.at[0, sl], p0b)
    pltpu.sync_copy(part_h.at[1, sl], p1b)
    pltpu.sync_copy(h0_h.at[sl], h0b)
    pltpu.sync_copy(ws0_h, wsb)

    def hloop(i, _):
        q = pl.ds(i * 16, 16)
        hab[q] = p0b[q] + p1b[q] + h0b[q] * wsb[...]
        p0b[q] = jnp.zeros((16,), _F32)
        return 0
    lax.fori_loop(0, NTS // 16, hloop, 0)
    pltpu.sync_copy(hab, h0a_s.at[sl])

    @pl.when(c == 0)
    def _():
        pltpu.sync_copy(hab, h0a_out.at[sl])

    pltpu.sync_copy(p0b, ag0.at[sl])
    pltpu.sync_copy(p0b, ag1.at[sl])
    pltpu.sync_copy(p0b, ag2.at[sl])
    pltpu.sync_copy(p0b, ag3.at[sl])
    plsc.subcore_barrier()

    def chunk(i, _):
        base = w * EPW + i * CHW
        ds = pl.ds(base, CHW)
        pltpu.sync_copy(src_h.at[ds], srcb)
        pltpu.sync_copy(dst_h.at[ds], dstb)
        pltpu.sync_copy(a1_h.at[ds], a1b)
        pltpu.sync_copy(b1_h.at[ds], b1b)
        pltpu.sync_copy(px_s.at[srcb], gxs)
        pltpu.sync_copy(py_s.at[srcb], gys)
        pltpu.sync_copy(pz_s.at[srcb], gzs)
        pltpu.sync_copy(px_s.at[dstb], gxd)
        pltpu.sync_copy(py_s.at[dstb], gyd)
        pltpu.sync_copy(pz_s.at[dstb], gzd)
        pltpu.sync_copy(h0a_s.at[srcb], hsb)

        def grp(g, _):
            q = pl.ds(g * 16, 16)
            dx = gxs[q] - gxd[q]
            dy = gys[q] - gyd[q]
            dz = gzs[q] - gzd[q]
            r = _newton_r(dx, dy, dz)
            qq = _f(1.0) / (r + _f(1e-8))
            hs = hsb[q]
            t = b1b[q] * hs * qq
            m0b[q] = a1b[q] * hs
            m1b[q] = t * dx
            m2b[q] = t * dy
            m3b[q] = t * dz
            return 0
        lax.fori_loop(0, NGRP, grp, 0)
        pltpu.sync_copy(m0b, ag0.at[dstb], add=True)
        pltpu.sync_copy(m1b, ag1.at[dstb], add=True)
        pltpu.sync_copy(m2b, ag2.at[dstb], add=True)
        pltpu.sync_copy(m3b, ag3.at[dstb], add=True)
        return 0

    lax.fori_loop(0, NCHUNK, chunk, 0)
    plsc.subcore_barrier()
    pltpu.sync_copy(ag0.at[sl], out_h.at[c, 0, sl])
    pltpu.sync_copy(ag1.at[sl], out_h.at[c, 1, sl])
    pltpu.sync_copy(ag2.at[sl], out_h.at[c, 2, sl])
    pltpu.sync_copy(ag3.at[sl], out_h.at[c, 3, sl])


def _layer1(src1, dst1, a1f, b1f, posx, posy, posz, h0p, part0, ws0v):
    f = pl.kernel(
        _k6_body,
        out_type=(jax.ShapeDtypeStruct((NT,), _F32),
                  jax.ShapeDtypeStruct((NC, 4, NT), _F32)),
        mesh=plsc.VectorSubcoreMesh(core_axis_name="c", subcore_axis_name="s"),
        compiler_params=pltpu.CompilerParams(needs_layout_passes=False),
        scratch_types=(
            [pltpu.VMEM_SHARED((NT,), _F32)] * 8
            + [pltpu.VMEM((CHW,), _I32)] * 2
            + [pltpu.VMEM((CHW,), _F32)] * 13
            + [pltpu.VMEM((NTS,), _F32)] * 4
            + [pltpu.VMEM((16,), _F32)]
        ),
    )
    return f(src1, dst1, a1f, b1f, posx, posy, posz, h0p, part0, ws0v)


def _k7_body(p0_ref, p1_ref, ha_ref, ws1_ref, o_ref):
    o = p0_ref[...] + p1_ref[...]
    row = lax.broadcasted_iota(_I32, o.shape, 0)
    ha = jnp.broadcast_to(ha_ref[...], o.shape)
    o_ref[...] = o + jnp.where(row == 0, ha * ws1_ref[0, 0], _f(0.0))


def _combine(p0, p1, h0a, ws1):
    bt = 12544
    blk = pl.BlockSpec((4, bt), lambda i: (0, i))
    return pl.pallas_call(
        _k7_body,
        grid=(NT // bt,),
        in_specs=[blk, blk,
                  pl.BlockSpec((1, bt), lambda i: (0, i)),
                  pl.BlockSpec(memory_space=pltpu.SMEM)],
        out_specs=blk,
        out_shape=jax.ShapeDtypeStruct((4, NT), _F32),
    )(p0, p1, h0a.reshape(1, NT), ws1)


def kernel(x, pos, edge_index, edge_attr, W1, b1, W2, b2,
           Wr1_0, br1_0, Wr2_0, br2_0, ws_0,
           Wr1_1, br1_1, Wr2_1, br2_1, ws_1):
    src1 = jnp.pad(edge_index[0].astype(_I32), (0, EPAD - E))
    dst1 = jnp.pad(edge_index[1].astype(_I32), (0, EPAD - E), constant_values=N)
    ea0 = jnp.pad(edge_attr[:, 0], (0, EPAD - E)).reshape(EROWS, LAN)
    ea1 = jnp.pad(edge_attr[:, 1], (0, EPAD - E)).reshape(EROWS, LAN)
    posx = jnp.pad(pos[:, 0], (0, NT - N))
    posy = jnp.pad(pos[:, 1], (0, NT - N))
    posz = jnp.pad(pos[:, 2], (0, NT - N))

    h0 = _node_mlp(x, W1, b1, W2, b2)
    h0p = jnp.pad(h0[:, 0], (0, NT - N))
    r1 = _edge_r(src1, dst1, posx, posy, posz)
    a0, a1, b1r = _radial(ea0, ea1, r1.reshape(EROWS, LAN),
                          Wr1_0, br1_0, Wr2_0, br2_0,
                          Wr1_1, br1_1, Wr2_1, br2_1)
    part0 = _layer0(src1, dst1, a0.reshape(EPAD), h0p)
    ws0v = jnp.broadcast_to(ws_0.reshape(1), (16,))
    h0a, part4 = _layer1(src1, dst1, a1.reshape(EPAD), b1r.reshape(EPAD),
                         posx, posy, posz, h0p, part0, ws0v)
    out4 = _combine(part4[0], part4[1], h0a, ws_1)
    return out4[:, :N].T


# speedup vs baseline: 76.9505x; 1.2341x over previous
import numpy as np

import jax
import jax.numpy as jnp
from jax import lax
from jax.experimental import pallas as pl
from jax.experimental.pallas import tpu as pltpu
from jax.experimental.pallas import tpu_sc as plsc

N = 100000
E = 3200000
D_IN = 67
H = 32

LAN = 128
NC = 2
NS = 16
NW = NC * NS
EPW = 100352
EROWS = 25088
EPAD = NW * EPW
NT = 100352
NTS = NT // NS

CH2 = 7168
CH4 = 14336
CH6 = 7168

_F32 = jnp.float32
_I32 = jnp.int32
_f = np.float32
_MAGIC = np.int32(0x5F3759DF)


def _newton_r(dx, dy, dz):
    rsq = dx * dx + dy * dy + dz * dz
    bits = plsc.bitcast(rsq, _I32)
    y = plsc.bitcast(_MAGIC - lax.shift_right_logical(bits, 1), _F32)
    hr = rsq * _f(0.5)
    for _ in range(3):
        y = y * (_f(1.5) - hr * y * y)
    return rsq * y


def _k1_body(x_ref, w1_ref, b1_ref, w2_ref, b2_ref, o_ref):
    xb = x_ref[...].astype(jnp.bfloat16)
    w1 = w1_ref[...].astype(jnp.bfloat16)
    l0 = jnp.dot(xb, w1, preferred_element_type=_F32)
    l0 = l0 + b1_ref[...]
    l0 = jnp.where(l0 > 0, l0, jnp.exp(l0) - _f(1.0))
    h = jnp.dot(l0.astype(jnp.bfloat16), w2_ref[...].astype(jnp.bfloat16),
                preferred_element_type=_F32) + b2_ref[0, 0]
    o_ref[...] = h


def _node_mlp(x, W1, b1, W2, b2):
    nb = 1000
    return pl.pallas_call(
        _k1_body,
        grid=(N // nb,),
        in_specs=[
            pl.BlockSpec((nb, D_IN), lambda i: (i, 0)),
            pl.BlockSpec((D_IN, D_IN), lambda i: (0, 0)),
            pl.BlockSpec((1, D_IN), lambda i: (0, 0)),
            pl.BlockSpec((D_IN, 1), lambda i: (0, 0)),
            pl.BlockSpec(memory_space=pltpu.SMEM),
        ],
        out_specs=pl.BlockSpec((nb, 1), lambda i: (i, 0)),
        out_shape=jax.ShapeDtypeStruct((N, 1), _F32),
    )(x, W1, b1.reshape(1, D_IN), W2, b2.reshape(1, 1))


def _k2_body(src_h, dst_h, px_h, py_h, pz_h,
             rx_h, ry_h, rz_h, r_h,
             px_s, py_s, pz_s,
             srcb, dstb, gxs, gys, gzs, gxd, gyd, gzd, rb):
    c = lax.axis_index("c")
    s = lax.axis_index("s")
    w = s * NC + c
    sl = pl.ds(s * NTS, NTS)
    pltpu.sync_copy(px_h.at[sl], px_s.at[sl])
    pltpu.sync_copy(py_h.at[sl], py_s.at[sl])
    pltpu.sync_copy(pz_h.at[sl], pz_s.at[sl])
    plsc.subcore_barrier()

    def chunk(i, _):
        base = w * EPW + i * CH2
        ds = pl.ds(base, CH2)
        pltpu.sync_copy(src_h.at[ds], srcb)
        pltpu.sync_copy(dst_h.at[ds], dstb)
        pltpu.sync_copy(px_s.at[srcb], gxs)
        pltpu.sync_copy(py_s.at[srcb], gys)
        pltpu.sync_copy(pz_s.at[srcb], gzs)
        pltpu.sync_copy(px_s.at[dstb], gxd)
        pltpu.sync_copy(py_s.at[dstb], gyd)
        pltpu.sync_copy(pz_s.at[dstb], gzd)

        def grp(g, _):
            q = pl.ds(g * 16, 16)
            dx = gxs[q] - gxd[q]
            dy = gys[q] - gyd[q]
            dz = gzs[q] - gzd[q]
            r = _newton_r(dx, dy, dz)
            qq = _f(1.0) / (r + _f(1e-8))
            rb[q] = r
            gxs[q] = dx * qq
            gys[q] = dy * qq
            gzs[q] = dz * qq
            return 0
        lax.fori_loop(0, CH2 // 16, grp, 0)
        pltpu.sync_copy(rb, r_h.at[ds])
        pltpu.sync_copy(gxs, rx_h.at[ds])
        pltpu.sync_copy(gys, ry_h.at[ds])
        pltpu.sync_copy(gzs, rz_h.at[ds])
        return 0

    lax.fori_loop(0, EPW // CH2, chunk, 0)


def _edge_r(src1, dst1, posx, posy, posz):
    f = pl.kernel(
        _k2_body,
        out_type=(jax.ShapeDtypeStruct((EPAD,), _F32),) * 4,
        mesh=plsc.VectorSubcoreMesh(core_axis_name="c", subcore_axis_name="s"),
        compiler_params=pltpu.CompilerParams(needs_layout_passes=False),
        scratch_types=(
            [pltpu.VMEM_SHARED((NT,), _F32)] * 3
            + [pltpu.VMEM((CH2,), _I32)] * 2
            + [pltpu.VMEM((CH2,), _F32)] * 7
        ),
    )
    return f(src1, dst1, posx, posy, posz)


def _k3_body(e0_ref, e1_ref, r_ref, w10, b10, w20, b20, w11, b11, w21, b21,
             a0_ref, a1_ref, b1_ref):
    def b16(v):
        return v.astype(jnp.bfloat16).astype(_F32)

    e0 = b16(e0_ref[...])
    e1 = b16(e1_ref[...])
    rr = b16(r_ref[...])
    acc0 = jnp.zeros(e0.shape, _F32)
    acc1 = jnp.zeros(e0.shape, _F32)
    accb = jnp.zeros(e0.shape, _F32)
    for j in range(H):
        h0 = e0 * b16(w10[0, j]) + e1 * b16(w10[1, j]) + rr * b16(w10[2, j])
        h0 = jnp.maximum(h0 + b10[0, j], _f(0.0))
        h0 = b16(h0)
        acc0 = acc0 + h0 * b16(w20[j, 0])
        h1 = e0 * b16(w11[0, j]) + e1 * b16(w11[1, j]) + rr * b16(w11[2, j])
        h1 = jnp.maximum(h1 + b11[0, j], _f(0.0))
        h1 = b16(h1)
        acc1 = acc1 + h1 * b16(w21[j, 0])
        accb = accb + h1 * b16(w21[j, 1])
    a0_ref[...] = acc0 + b20[0, 0]
    a1_ref[...] = acc1 + b21[0, 0]
    b1_ref[...] = accb + b21[0, 1]


def _radial(ea0, ea1, r2, Wr1_0, br1_0, Wr2_0, br2_0, Wr1_1, br1_1, Wr2_1, br2_1):
    rb = 512
    smem = pl.BlockSpec(memory_space=pltpu.SMEM)
    blk = pl.BlockSpec((rb, LAN), lambda i: (i, 0))
    return pl.pallas_call(
        _k3_body,
        grid=(EROWS // rb,),
        in_specs=[blk, blk, blk] + [smem] * 8,
        out_specs=[blk, blk, blk],
        out_shape=[jax.ShapeDtypeStruct((EROWS, LAN), _F32)] * 3,
    )(ea0, ea1, r2,
      Wr1_0, br1_0.reshape(1, H), Wr2_0, br2_0.reshape(1, 2),
      Wr1_1, br1_1.reshape(1, H), Wr2_1, br2_1.reshape(1, 2))


def _k4_body(src_h, dst_h, a0_h, h0_h, out_h,
             h0_s, agg_s,
             srcb, dstb, a0b, hsb, mb):
    c = lax.axis_index("c")
    s = lax.axis_index("s")
    w = s * NC + c
    sl = pl.ds(s * NTS, NTS)
    pltpu.sync_copy(h0_h.at[sl], h0_s.at[sl])

    def zloop(i, _):
        mb[pl.ds(i * 16, 16)] = jnp.zeros((16,), _F32)
        return 0
    lax.fori_loop(0, NTS // 16, zloop, 0)
    pltpu.sync_copy(mb.at[pl.ds(0, NTS)], agg_s.at[sl])
    plsc.subcore_barrier()

    def chunk(i, _):
        base = w * EPW + i * CH4
        ds = pl.ds(base, CH4)
        pltpu.sync_copy(src_h.at[ds], srcb)
        pltpu.sync_copy(dst_h.at[ds], dstb)
        pltpu.sync_copy(a0_h.at[ds], a0b)
        pltpu.sync_copy(h0_s.at[srcb], hsb)

        def grp(g, _):
            q = pl.ds(g * 16, 16)
            mb[q] = a0b[q] * hsb[q]
            return 0
        lax.fori_loop(0, CH4 // 16, grp, 0)
        pltpu.sync_copy(mb, agg_s.at[dstb], add=True)
        return 0

    lax.fori_loop(0, EPW // CH4, chunk, 0)
    plsc.subcore_barrier()
    pltpu.sync_copy(agg_s.at[sl], out_h.at[c, sl])


def _layer0(src1, dst1, a0f, h0p):
    f = pl.kernel(
        _k4_body,
        out_type=jax.ShapeDtypeStruct((NC, NT), _F32),
        mesh=plsc.VectorSubcoreMesh(core_axis_name="c", subcore_axis_name="s"),
        compiler_params=pltpu.CompilerParams(needs_layout_passes=False),
        scratch_types=(
            [pltpu.VMEM_SHARED((NT,), _F32)] * 2
            + [pltpu.VMEM((CH4,), _I32)] * 2
            + [pltpu.VMEM((CH4,), _F32)] * 3
        ),
    )
    return f(src1, dst1, a0f, h0p)


def _k6_body(src_h, dst_h, a1_h, b1_h, rx_h, ry_h, rz_h, h0_h, part_h, ws0_h,
             h0a_out, out_h,
             h0a_s, ag0, ag1, ag2, ag3,
             srcb, dstb, a1b, b1b, rxb, ryb, rzb, hsb,
             m0b, m1b, m2b, m3b, wsb):
    c = lax.axis_index("c")
    s = lax.axis_index("s")
    w = s * NC + c
    sl = pl.ds(s * NTS, NTS)
    nts = pl.ds(0, NTS)
    pltpu.sync_copy(part_h.at[0, sl], a1b.at[nts])
    pltpu.sync_copy(part_h.at[1, sl], b1b.at[nts])
    pltpu.sync_copy(h0_h.at[sl], rxb.at[nts])
    pltpu.sync_copy(ws0_h, wsb)

    def hloop(i, _):
        q = pl.ds(i * 16, 16)
        ryb[q] = a1b[q] + b1b[q] + rxb[q] * wsb[...]
        rzb[q] = jnp.zeros((16,), _F32)
        return 0
    lax.fori_loop(0, NTS // 16, hloop, 0)
    pltpu.sync_copy(ryb.at[nts], h0a_s.at[sl])

    @pl.when(c == 0)
    def _():
        pltpu.sync_copy(ryb.at[nts], h0a_out.at[sl])

    pltpu.sync_copy(rzb.at[nts], ag0.at[sl])
    pltpu.sync_copy(rzb.at[nts], ag1.at[sl])
    pltpu.sync_copy(rzb.at[nts], ag2.at[sl])
    pltpu.sync_copy(rzb.at[nts], ag3.at[sl])
    plsc.subcore_barrier()

    def chunk(i, _):
        base = w * EPW + i * CH6
        ds = pl.ds(base, CH6)
        pltpu.sync_copy(src_h.at[ds], srcb)
        pltpu.sync_copy(dst_h.at[ds], dstb)
        pltpu.sync_copy(a1_h.at[ds], a1b)
        pltpu.sync_copy(b1_h.at[ds], b1b)
        pltpu.sync_copy(rx_h.at[ds], rxb)
        pltpu.sync_copy(ry_h.at[ds], ryb)
        pltpu.sync_copy(rz_h.at[ds], rzb)
        pltpu.sync_copy(h0a_s.at[srcb], hsb)

        def grp(g, _):
            q = pl.ds(g * 16, 16)
            hs = hsb[q]
            t = b1b[q] * hs
            m0b[q] = a1b[q] * hs
            m1b[q] = t * rxb[q]
            m2b[q] = t * ryb[q]
            m3b[q] = t * rzb[q]
            return 0
        lax.fori_loop(0, CH6 // 16, grp, 0)
        pltpu.sync_copy(m0b, ag0.at[dstb], add=True)
        pltpu.sync_copy(m1b, ag1.at[dstb], add=True)
        pltpu.sync_copy(m2b, ag2.at[dstb], add=True)
        pltpu.sync_copy(m3b, ag3.at[dstb], add=True)
        return 0

    lax.fori_loop(0, EPW // CH6, chunk, 0)
    plsc.subcore_barrier()
    pltpu.sync_copy(ag0.at[sl], out_h.at[c, 0, sl])
    pltpu.sync_copy(ag1.at[sl], out_h.at[c, 1, sl])
    pltpu.sync_copy(ag2.at[sl], out_h.at[c, 2, sl])
    pltpu.sync_copy(ag3.at[sl], out_h.at[c, 3, sl])


def _layer1(src1, dst1, a1f, b1f, rx, ry, rz, h0p, part0, ws0v):
    f = pl.kernel(
        _k6_body,
        out_type=(jax.ShapeDtypeStruct((NT,), _F32),
                  jax.ShapeDtypeStruct((NC, 4, NT), _F32)),
        mesh=plsc.VectorSubcoreMesh(core_axis_name="c", subcore_axis_name="s"),
        compiler_params=pltpu.CompilerParams(needs_layout_passes=False),
        scratch_types=(
            [pltpu.VMEM_SHARED((NT,), _F32)] * 5
            + [pltpu.VMEM((CH6,), _I32)] * 2
            + [pltpu.VMEM((CH6,), _F32)] * 10
            + [pltpu.VMEM((16,), _F32)]
        ),
    )
    return f(src1, dst1, a1f, b1f, rx, ry, rz, h0p, part0, ws0v)


def _k7_body(p0_ref, p1_ref, ha_ref, ws1_ref, o_ref):
    o = p0_ref[...] + p1_ref[...]
    row = lax.broadcasted_iota(_I32, o.shape, 0)
    ha = jnp.broadcast_to(ha_ref[...], o.shape)
    o_ref[...] = o + jnp.where(row == 0, ha * ws1_ref[0, 0], _f(0.0))


def _combine(p0, p1, h0a, ws1):
    bt = 12544
    blk = pl.BlockSpec((4, bt), lambda i: (0, i))
    return pl.pallas_call(
        _k7_body,
        grid=(NT // bt,),
        in_specs=[blk, blk,
                  pl.BlockSpec((1, bt), lambda i: (0, i)),
                  pl.BlockSpec(memory_space=pltpu.SMEM)],
        out_specs=blk,
        out_shape=jax.ShapeDtypeStruct((4, NT), _F32),
    )(p0, p1, h0a.reshape(1, NT), ws1)


def kernel(x, pos, edge_index, edge_attr, W1, b1, W2, b2,
           Wr1_0, br1_0, Wr2_0, br2_0, ws_0,
           Wr1_1, br1_1, Wr2_1, br2_1, ws_1):
    src1 = jnp.pad(edge_index[0].astype(_I32), (0, EPAD - E))
    dst1 = jnp.pad(edge_index[1].astype(_I32), (0, EPAD - E), constant_values=N)
    ea0 = jnp.pad(edge_attr[:, 0], (0, EPAD - E)).reshape(EROWS, LAN)
    ea1 = jnp.pad(edge_attr[:, 1], (0, EPAD - E)).reshape(EROWS, LAN)
    posx = jnp.pad(pos[:, 0], (0, NT - N))
    posy = jnp.pad(pos[:, 1], (0, NT - N))
    posz = jnp.pad(pos[:, 2], (0, NT - N))

    h0 = _node_mlp(x, W1, b1, W2, b2)
    h0p = jnp.pad(h0[:, 0], (0, NT - N))
    rx, ry, rz, r1 = _edge_r(src1, dst1, posx, posy, posz)
    a0, a1, b1r = _radial(ea0, ea1, r1.reshape(EROWS, LAN),
                          Wr1_0, br1_0, Wr2_0, br2_0,
                          Wr1_1, br1_1, Wr2_1, br2_1)
    part0 = _layer0(src1, dst1, a0.reshape(EPAD), h0p)
    ws0v = jnp.broadcast_to(ws_0.reshape(1), (16,))
    h0a, part4 = _layer1(src1, dst1, a1.reshape(EPAD), b1r.reshape(EPAD),
                         rx, ry, rz, h0p, part0, ws0v)
    out4 = _combine(part4[0], part4[1], h0a, ws_1)
    return out4[:, :N].T


# speedup vs baseline: 81.8013x; 1.0630x over previous
import numpy as np

import jax
import jax.numpy as jnp
from jax import lax
from jax.experimental import pallas as pl
from jax.experimental.pallas import tpu as pltpu
from jax.experimental.pallas import tpu_sc as plsc

N = 100000
E = 3200000
D_IN = 67
H = 32

LAN = 128
NC = 2
NS = 16
NW = NC * NS
EPW = 100352
EROWS = 25088
EPAD = NW * EPW
NT = 100352
NTS = NT // NS

CH2 = 7168
CH4 = 14336
CH6 = 7168

_F32 = jnp.float32
_I32 = jnp.int32
_f = np.float32
_MAGIC = np.int32(0x5F3759DF)


def _newton_r(dx, dy, dz):
    rsq = dx * dx + dy * dy + dz * dz
    bits = plsc.bitcast(rsq, _I32)
    y = plsc.bitcast(_MAGIC - lax.shift_right_logical(bits, 1), _F32)
    hr = rsq * _f(0.5)
    for _ in range(3):
        y = y * (_f(1.5) - hr * y * y)
    return rsq * y


def _drain(descs):
    for d in descs:
        d.wait()


def _k1_body(x_ref, w1_ref, b1_ref, w2_ref, b2_ref, o_ref):
    xb = x_ref[...].astype(jnp.bfloat16)
    w1 = w1_ref[...].astype(jnp.bfloat16)
    l0 = jnp.dot(xb, w1, preferred_element_type=_F32)
    l0 = l0 + b1_ref[...]
    l0 = jnp.where(l0 > 0, l0, jnp.exp(l0) - _f(1.0))
    h = jnp.dot(l0.astype(jnp.bfloat16), w2_ref[...].astype(jnp.bfloat16),
                preferred_element_type=_F32) + b2_ref[0, 0]
    o_ref[...] = h


def _node_mlp(x, W1, b1, W2, b2):
    nb = 1000
    return pl.pallas_call(
        _k1_body,
        grid=(N // nb,),
        in_specs=[
            pl.BlockSpec((nb, D_IN), lambda i: (i, 0)),
            pl.BlockSpec((D_IN, D_IN), lambda i: (0, 0)),
            pl.BlockSpec((1, D_IN), lambda i: (0, 0)),
            pl.BlockSpec((D_IN, 1), lambda i: (0, 0)),
            pl.BlockSpec(memory_space=pltpu.SMEM),
        ],
        out_specs=pl.BlockSpec((nb, 1), lambda i: (i, 0)),
        out_shape=jax.ShapeDtypeStruct((N, 1), _F32),
    )(x, W1, b1.reshape(1, D_IN), W2, b2.reshape(1, 1))


def _k2_body(src_h, dst_h, px_h, py_h, pz_h,
             rx_h, ry_h, rz_h, r_h,
             px_s, py_s, pz_s,
             srcb, dstb, gxs, gys, gzs, gxd, gyd, gzd, rb, sem):
    c = lax.axis_index("c")
    s = lax.axis_index("s")
    w = s * NC + c
    sl = pl.ds(s * NTS, NTS)
    _drain([pltpu.async_copy(px_h.at[sl], px_s.at[sl], sem),
            pltpu.async_copy(py_h.at[sl], py_s.at[sl], sem),
            pltpu.async_copy(pz_h.at[sl], pz_s.at[sl], sem)])
    plsc.subcore_barrier()

    def chunk(i, _):
        base = w * EPW + i * CH2
        ds = pl.ds(base, CH2)
        _drain([pltpu.async_copy(src_h.at[ds], srcb, sem),
                pltpu.async_copy(dst_h.at[ds], dstb, sem)])
        _drain([pltpu.async_copy(px_s.at[srcb], gxs, sem),
                pltpu.async_copy(py_s.at[srcb], gys, sem),
                pltpu.async_copy(pz_s.at[srcb], gzs, sem),
                pltpu.async_copy(px_s.at[dstb], gxd, sem),
                pltpu.async_copy(py_s.at[dstb], gyd, sem),
                pltpu.async_copy(pz_s.at[dstb], gzd, sem)])

        def grp(g, _):
            q = pl.ds(g * 16, 16)
            dx = gxs[q] - gxd[q]
            dy = gys[q] - gyd[q]
            dz = gzs[q] - gzd[q]
            r = _newton_r(dx, dy, dz)
            qq = _f(1.0) / (r + _f(1e-8))
            rb[q] = r
            gxs[q] = dx * qq
            gys[q] = dy * qq
            gzs[q] = dz * qq
            return 0
        lax.fori_loop(0, CH2 // 16, grp, 0)
        _drain([pltpu.async_copy(rb, r_h.at[ds], sem),
                pltpu.async_copy(gxs, rx_h.at[ds], sem),
                pltpu.async_copy(gys, ry_h.at[ds], sem),
                pltpu.async_copy(gzs, rz_h.at[ds], sem)])
        return 0

    lax.fori_loop(0, EPW // CH2, chunk, 0)


def _edge_r(src1, dst1, posx, posy, posz):
    f = pl.kernel(
        _k2_body,
        out_type=(jax.ShapeDtypeStruct((EPAD,), _F32),) * 4,
        mesh=plsc.VectorSubcoreMesh(core_axis_name="c", subcore_axis_name="s"),
        compiler_params=pltpu.CompilerParams(needs_layout_passes=False),
        scratch_types=(
            [pltpu.VMEM_SHARED((NT,), _F32)] * 3
            + [pltpu.VMEM((CH2,), _I32)] * 2
            + [pltpu.VMEM((CH2,), _F32)] * 7
            + [pltpu.SemaphoreType.DMA]
        ),
    )
    return f(src1, dst1, posx, posy, posz)


def _k3_body(e0_ref, e1_ref, r_ref, w10, b10, w20, b20, w11, b11, w21, b21,
             a0_ref, a1_ref, b1_ref):
    def b16(v):
        return v.astype(jnp.bfloat16).astype(_F32)

    e0 = b16(e0_ref[...])
    e1 = b16(e1_ref[...])
    rr = b16(r_ref[...])
    acc0 = jnp.zeros(e0.shape, _F32)
    acc1 = jnp.zeros(e0.shape, _F32)
    accb = jnp.zeros(e0.shape, _F32)
    for j in range(H):
        h0 = e0 * b16(w10[0, j]) + e1 * b16(w10[1, j]) + rr * b16(w10[2, j])
        h0 = jnp.maximum(h0 + b10[0, j], _f(0.0))
        h0 = b16(h0)
        acc0 = acc0 + h0 * b16(w20[j, 0])
        h1 = e0 * b16(w11[0, j]) + e1 * b16(w11[1, j]) + rr * b16(w11[2, j])
        h1 = jnp.maximum(h1 + b11[0, j], _f(0.0))
        h1 = b16(h1)
        acc1 = acc1 + h1 * b16(w21[j, 0])
        accb = accb + h1 * b16(w21[j, 1])
    a0_ref[...] = acc0 + b20[0, 0]
    a1_ref[...] = acc1 + b21[0, 0]
    b1_ref[...] = accb + b21[0, 1]


def _radial(ea0, ea1, r2, Wr1_0, br1_0, Wr2_0, br2_0, Wr1_1, br1_1, Wr2_1, br2_1):
    rb = 512
    smem = pl.BlockSpec(memory_space=pltpu.SMEM)
    blk = pl.BlockSpec((rb, LAN), lambda i: (i, 0))
    return pl.pallas_call(
        _k3_body,
        grid=(EROWS // rb,),
        in_specs=[blk, blk, blk] + [smem] * 8,
        out_specs=[blk, blk, blk],
        out_shape=[jax.ShapeDtypeStruct((EROWS, LAN), _F32)] * 3,
    )(ea0, ea1, r2,
      Wr1_0, br1_0.reshape(1, H), Wr2_0, br2_0.reshape(1, 2),
      Wr1_1, br1_1.reshape(1, H), Wr2_1, br2_1.reshape(1, 2))


def _k4_body(src_h, dst_h, a0_h, h0_h, out_h,
             h0_s, agg_s,
             srcb, dstb, a0b, hsb, mb, sem):
    c = lax.axis_index("c")
    s = lax.axis_index("s")
    w = s * NC + c
    sl = pl.ds(s * NTS, NTS)
    pltpu.sync_copy(h0_h.at[sl], h0_s.at[sl])

    def zloop(i, _):
        mb[pl.ds(i * 16, 16)] = jnp.zeros((16,), _F32)
        return 0
    lax.fori_loop(0, NTS // 16, zloop, 0)
    pltpu.sync_copy(mb.at[pl.ds(0, NTS)], agg_s.at[sl])
    plsc.subcore_barrier()

    def chunk(i, _):
        base = w * EPW + i * CH4
        ds = pl.ds(base, CH4)
        _drain([pltpu.async_copy(src_h.at[ds], srcb, sem),
                pltpu.async_copy(dst_h.at[ds], dstb, sem),
                pltpu.async_copy(a0_h.at[ds], a0b, sem)])
        pltpu.sync_copy(h0_s.at[srcb], hsb)

        def grp(g, _):
            q = pl.ds(g * 16, 16)
            mb[q] = a0b[q] * hsb[q]
            return 0
        lax.fori_loop(0, CH4 // 16, grp, 0)
        pltpu.sync_copy(mb, agg_s.at[dstb], add=True)
        return 0

    lax.fori_loop(0, EPW // CH4, chunk, 0)
    plsc.subcore_barrier()
    pltpu.sync_copy(agg_s.at[sl], out_h.at[c, sl])


def _layer0(src1, dst1, a0f, h0p):
    f = pl.kernel(
        _k4_body,
        out_type=jax.ShapeDtypeStruct((NC, NT), _F32),
        mesh=plsc.VectorSubcoreMesh(core_axis_name="c", subcore_axis_name="s"),
        compiler_params=pltpu.CompilerParams(needs_layout_passes=False),
        scratch_types=(
            [pltpu.VMEM_SHARED((NT,), _F32)] * 2
            + [pltpu.VMEM((CH4,), _I32)] * 2
            + [pltpu.VMEM((CH4,), _F32)] * 3
            + [pltpu.SemaphoreType.DMA]
        ),
    )
    return f(src1, dst1, a0f, h0p)


def _k6_body(src_h, dst_h, a1_h, b1_h, rx_h, ry_h, rz_h, h0_h, part_h, ws0_h,
             h0a_out, out_h,
             h0a_s, ag0, ag1, ag2, ag3,
             srcb, dstb, a1b, b1b, rxb, ryb, rzb, hsb,
             m0b, m1b, m2b, m3b, wsb, sem):
    c = lax.axis_index("c")
    s = lax.axis_index("s")
    w = s * NC + c
    sl = pl.ds(s * NTS, NTS)
    nts = pl.ds(0, NTS)
    _drain([pltpu.async_copy(part_h.at[0, sl], a1b.at[nts], sem),
            pltpu.async_copy(part_h.at[1, sl], b1b.at[nts], sem),
            pltpu.async_copy(h0_h.at[sl], rxb.at[nts], sem),
            pltpu.async_copy(ws0_h, wsb, sem)])

    def hloop(i, _):
        q = pl.ds(i * 16, 16)
        ryb[q] = a1b[q] + b1b[q] + rxb[q] * wsb[...]
        rzb[q] = jnp.zeros((16,), _F32)
        return 0
    lax.fori_loop(0, NTS // 16, hloop, 0)
    pltpu.sync_copy(ryb.at[nts], h0a_s.at[sl])

    @pl.when(c == 0)
    def _():
        pltpu.sync_copy(ryb.at[nts], h0a_out.at[sl])

    _drain([pltpu.async_copy(rzb.at[nts], ag0.at[sl], sem),
            pltpu.async_copy(rzb.at[nts], ag1.at[sl], sem),
            pltpu.async_copy(rzb.at[nts], ag2.at[sl], sem),
            pltpu.async_copy(rzb.at[nts], ag3.at[sl], sem)])
    plsc.subcore_barrier()

    def chunk(i, _):
        base = w * EPW + i * CH6
        ds = pl.ds(base, CH6)
        _drain([pltpu.async_copy(src_h.at[ds], srcb, sem),
                pltpu.async_copy(dst_h.at[ds], dstb, sem),
                pltpu.async_copy(a1_h.at[ds], a1b, sem),
                pltpu.async_copy(b1_h.at[ds], b1b, sem),
                pltpu.async_copy(rx_h.at[ds], rxb, sem),
                pltpu.async_copy(ry_h.at[ds], ryb, sem),
                pltpu.async_copy(rz_h.at[ds], rzb, sem)])
        pltpu.sync_copy(h0a_s.at[srcb], hsb)

        def grp(g, _):
            q = pl.ds(g * 16, 16)
            hs = hsb[q]
            t = b1b[q] * hs
            m0b[q] = a1b[q] * hs
            m1b[q] = t * rxb[q]
            m2b[q] = t * ryb[q]
            m3b[q] = t * rzb[q]
            return 0
        lax.fori_loop(0, CH6 // 16, grp, 0)
        _drain([pltpu.async_copy(m0b, ag0.at[dstb], sem, add=True),
                pltpu.async_copy(m1b, ag1.at[dstb], sem, add=True),
                pltpu.async_copy(m2b, ag2.at[dstb], sem, add=True),
                pltpu.async_copy(m3b, ag3.at[dstb], sem, add=True)])
        return 0

    lax.fori_loop(0, EPW // CH6, chunk, 0)
    plsc.subcore_barrier()
    _drain([pltpu.async_copy(ag0.at[sl], out_h.at[c, 0, sl], sem),
            pltpu.async_copy(ag1.at[sl], out_h.at[c, 1, sl], sem),
            pltpu.async_copy(ag2.at[sl], out_h.at[c, 2, sl], sem),
            pltpu.async_copy(ag3.at[sl], out_h.at[c, 3, sl], sem)])


def _layer1(src1, dst1, a1f, b1f, rx, ry, rz, h0p, part0, ws0v):
    f = pl.kernel(
        _k6_body,
        out_type=(jax.ShapeDtypeStruct((NT,), _F32),
                  jax.ShapeDtypeStruct((NC, 4, NT), _F32)),
        mesh=plsc.VectorSubcoreMesh(core_axis_name="c", subcore_axis_name="s"),
        compiler_params=pltpu.CompilerParams(needs_layout_passes=False),
        scratch_types=(
            [pltpu.VMEM_SHARED((NT,), _F32)] * 5
            + [pltpu.VMEM((CH6,), _I32)] * 2
            + [pltpu.VMEM((CH6,), _F32)] * 10
            + [pltpu.VMEM((16,), _F32)]
            + [pltpu.SemaphoreType.DMA]
        ),
    )
    return f(src1, dst1, a1f, b1f, rx, ry, rz, h0p, part0, ws0v)


def _k7_body(p0_ref, p1_ref, ha_ref, ws1_ref, o_ref):
    o = p0_ref[...] + p1_ref[...]
    row = lax.broadcasted_iota(_I32, o.shape, 0)
    ha = jnp.broadcast_to(ha_ref[...], o.shape)
    o_ref[...] = o + jnp.where(row == 0, ha * ws1_ref[0, 0], _f(0.0))


def _combine(p0, p1, h0a, ws1):
    bt = 12544
    blk = pl.BlockSpec((4, bt), lambda i: (0, i))
    return pl.pallas_call(
        _k7_body,
        grid=(NT // bt,),
        in_specs=[blk, blk,
                  pl.BlockSpec((1, bt), lambda i: (0, i)),
                  pl.BlockSpec(memory_space=pltpu.SMEM)],
        out_specs=blk,
        out_shape=jax.ShapeDtypeStruct((4, NT), _F32),
    )(p0, p1, h0a.reshape(1, NT), ws1)


def kernel(x, pos, edge_index, edge_attr, W1, b1, W2, b2,
           Wr1_0, br1_0, Wr2_0, br2_0, ws_0,
           Wr1_1, br1_1, Wr2_1, br2_1, ws_1):
    src1 = jnp.pad(edge_index[0].astype(_I32), (0, EPAD - E))
    dst1 = jnp.pad(edge_index[1].astype(_I32), (0, EPAD - E), constant_values=N)
    ea0 = jnp.pad(edge_attr[:, 0], (0, EPAD - E)).reshape(EROWS, LAN)
    ea1 = jnp.pad(edge_attr[:, 1], (0, EPAD - E)).reshape(EROWS, LAN)
    posx = jnp.pad(pos[:, 0], (0, NT - N))
    posy = jnp.pad(pos[:, 1], (0, NT - N))
    posz = jnp.pad(pos[:, 2], (0, NT - N))

    h0 = _node_mlp(x, W1, b1, W2, b2)
    h0p = jnp.pad(h0[:, 0], (0, NT - N))
    rx, ry, rz, r1 = _edge_r(src1, dst1, posx, posy, posz)
    a0, a1, b1r = _radial(ea0, ea1, r1.reshape(EROWS, LAN),
                          Wr1_0, br1_0, Wr2_0, br2_0,
                          Wr1_1, br1_1, Wr2_1, br2_1)
    part0 = _layer0(src1, dst1, a0.reshape(EPAD), h0p)
    ws0v = jnp.broadcast_to(ws_0.reshape(1), (16,))
    h0a, part4 = _layer1(src1, dst1, a1.reshape(EPAD), b1r.reshape(EPAD),
                         rx, ry, rz, h0p, part0, ws0v)
    out4 = _combine(part4[0], part4[1], h0a, ws_1)
    return out4[:, :N].T


# speedup vs baseline: 86.6446x; 1.0592x over previous
import numpy as np

import jax
import jax.numpy as jnp
from jax import lax
from jax.experimental import pallas as pl
from jax.experimental.pallas import tpu as pltpu
from jax.experimental.pallas import tpu_sc as plsc

N = 100000
E = 3200000
D_IN = 67
H = 32

LAN = 128
NC = 2
NS = 16
NW = NC * NS
EPW = 100352
EROWS = 25088
EPAD = NW * EPW
NT = 100352
NTS = NT // NS

CH2 = 7168
CH4 = 14336
CH6 = 7168

_F32 = jnp.float32
_I32 = jnp.int32
_f = np.float32
_MAGIC = np.int32(0x5F3759DF)


def _newton_r(dx, dy, dz):
    rsq = dx * dx + dy * dy + dz * dz
    bits = plsc.bitcast(rsq, _I32)
    y = plsc.bitcast(_MAGIC - lax.shift_right_logical(bits, 1), _F32)
    hr = rsq * _f(0.5)
    for _ in range(3):
        y = y * (_f(1.5) - hr * y * y)
    return rsq * y


def _drain(descs):
    for d in descs:
        d.wait()


def _k1_body(x_ref, w1_ref, b1_ref, w2_ref, b2_ref, o_ref):
    xb = x_ref[...].astype(jnp.bfloat16)
    w1 = w1_ref[...].astype(jnp.bfloat16)
    l0 = jnp.dot(xb, w1, preferred_element_type=_F32)
    l0 = l0 + b1_ref[...]
    l0 = jnp.where(l0 > 0, l0, jnp.exp(l0) - _f(1.0))
    h = jnp.dot(l0.astype(jnp.bfloat16), w2_ref[...].astype(jnp.bfloat16),
                preferred_element_type=_F32) + b2_ref[0, 0]
    o_ref[...] = h


def _node_mlp(x, W1, b1, W2, b2):
    nb = 1000
    return pl.pallas_call(
        _k1_body,
        grid=(N // nb,),
        in_specs=[
            pl.BlockSpec((nb, D_IN), lambda i: (i, 0)),
            pl.BlockSpec((D_IN, D_IN), lambda i: (0, 0)),
            pl.BlockSpec((1, D_IN), lambda i: (0, 0)),
            pl.BlockSpec((D_IN, 1), lambda i: (0, 0)),
            pl.BlockSpec(memory_space=pltpu.SMEM),
        ],
        out_specs=pl.BlockSpec((nb, 1), lambda i: (i, 0)),
        out_shape=jax.ShapeDtypeStruct((N, 1), _F32),
    )(x, W1, b1.reshape(1, D_IN), W2, b2.reshape(1, 1))


def _k2_body(src_h, dst_h, pxy_h, pz_h,
             rx_h, ry_h, rz_h, r_h,
             pxy_s, pz_s,
             srcb, dstb, gxys, gzs, gxyd, gzd, gys, rb, sem):
    c = lax.axis_index("c")
    s = lax.axis_index("s")
    w = s * NC + c
    sl = pl.ds(s * NTS, NTS)
    _drain([pltpu.async_copy(pxy_h.at[sl], pxy_s.at[sl], sem),
            pltpu.async_copy(pz_h.at[sl], pz_s.at[sl], sem)])
    plsc.subcore_barrier()

    def chunk(i, _):
        base = w * EPW + i * CH2
        ds = pl.ds(base, CH2)
        _drain([pltpu.async_copy(src_h.at[ds], srcb, sem),
                pltpu.async_copy(dst_h.at[ds], dstb, sem)])
        _drain([pltpu.async_copy(pxy_s.at[srcb], gxys, sem),
                pltpu.async_copy(pz_s.at[srcb], gzs, sem),
                pltpu.async_copy(pxy_s.at[dstb], gxyd, sem),
                pltpu.async_copy(pz_s.at[dstb], gzd, sem)])

        def grp(g, _):
            q = pl.ds(g * 16, 16)
            dxy = plsc.bitcast(gxys[q], jnp.bfloat16) - plsc.bitcast(gxyd[q], jnp.bfloat16)
            dx, dy = plsc.unpack(dxy, format=plsc.PackFormat.INTERLEAVED)
            dz = gzs[q] - gzd[q]
            r = _newton_r(dx, dy, dz)
            qq = _f(1.0) / (r + _f(1e-8))
            rb[q] = r
            gxys[q] = dx * qq
            gys[q] = dy * qq
            gzs[q] = dz * qq
            return 0
        lax.fori_loop(0, CH2 // 16, grp, 0)
        _drain([pltpu.async_copy(rb, r_h.at[ds], sem),
                pltpu.async_copy(gxys, rx_h.at[ds], sem),
                pltpu.async_copy(gys, ry_h.at[ds], sem),
                pltpu.async_copy(gzs, rz_h.at[ds], sem)])
        return 0

    lax.fori_loop(0, EPW // CH2, chunk, 0)


def _edge_r(src1, dst1, pxy, posz):
    f = pl.kernel(
        _k2_body,
        out_type=(jax.ShapeDtypeStruct((EPAD,), _F32),) * 4,
        mesh=plsc.VectorSubcoreMesh(core_axis_name="c", subcore_axis_name="s"),
        compiler_params=pltpu.CompilerParams(needs_layout_passes=False),
        scratch_types=(
            [pltpu.VMEM_SHARED((NT,), _F32)] * 2
            + [pltpu.VMEM((CH2,), _I32)] * 2
            + [pltpu.VMEM((CH2,), _F32)] * 6
            + [pltpu.SemaphoreType.DMA]
        ),
    )
    return f(src1, dst1, pxy, posz)


def _k3_body(e0_ref, e1_ref, r_ref, w10, b10, w20, b20, w11, b11, w21, b21,
             a0_ref, a1_ref, b1_ref):
    def b16(v):
        return v.astype(jnp.bfloat16).astype(_F32)

    e0 = b16(e0_ref[...])
    e1 = b16(e1_ref[...])
    rr = b16(r_ref[...])
    acc0 = jnp.zeros(e0.shape, _F32)
    acc1 = jnp.zeros(e0.shape, _F32)
    accb = jnp.zeros(e0.shape, _F32)
    for j in range(H):
        h0 = e0 * b16(w10[0, j]) + e1 * b16(w10[1, j]) + rr * b16(w10[2, j])
        h0 = jnp.maximum(h0 + b10[0, j], _f(0.0))
        h0 = b16(h0)
        acc0 = acc0 + h0 * b16(w20[j, 0])
        h1 = e0 * b16(w11[0, j]) + e1 * b16(w11[1, j]) + rr * b16(w11[2, j])
        h1 = jnp.maximum(h1 + b11[0, j], _f(0.0))
        h1 = b16(h1)
        acc1 = acc1 + h1 * b16(w21[j, 0])
        accb = accb + h1 * b16(w21[j, 1])
    a0_ref[...] = acc0 + b20[0, 0]
    a1_ref[...] = acc1 + b21[0, 0]
    b1_ref[...] = accb + b21[0, 1]


def _radial(ea0, ea1, r2, Wr1_0, br1_0, Wr2_0, br2_0, Wr1_1, br1_1, Wr2_1, br2_1):
    rb = 512
    smem = pl.BlockSpec(memory_space=pltpu.SMEM)
    blk = pl.BlockSpec((rb, LAN), lambda i: (i, 0))
    return pl.pallas_call(
        _k3_body,
        grid=(EROWS // rb,),
        in_specs=[blk, blk, blk] + [smem] * 8,
        out_specs=[blk, blk, blk],
        out_shape=[jax.ShapeDtypeStruct((EROWS, LAN), _F32)] * 3,
    )(ea0, ea1, r2,
      Wr1_0, br1_0.reshape(1, H), Wr2_0, br2_0.reshape(1, 2),
      Wr1_1, br1_1.reshape(1, H), Wr2_1, br2_1.reshape(1, 2))


def _k4_body(src_h, dst_h, a0_h, h0_h, out_h,
             h0_s, agg_s,
             srcb, dstb, a0b, hsb, mb, sem):
    c = lax.axis_index("c")
    s = lax.axis_index("s")
    w = s * NC + c
    sl = pl.ds(s * NTS, NTS)
    pltpu.sync_copy(h0_h.at[sl], h0_s.at[sl])

    def zloop(i, _):
        mb[pl.ds(i * 16, 16)] = jnp.zeros((16,), _F32)
        return 0
    lax.fori_loop(0, NTS // 16, zloop, 0)
    pltpu.sync_copy(mb.at[pl.ds(0, NTS)], agg_s.at[sl])
    plsc.subcore_barrier()

    def chunk(i, _):
        base = w * EPW + i * CH4
        ds = pl.ds(base, CH4)
        _drain([pltpu.async_copy(src_h.at[ds], srcb, sem),
                pltpu.async_copy(dst_h.at[ds], dstb, sem),
                pltpu.async_copy(a0_h.at[ds], a0b, sem)])
        pltpu.sync_copy(h0_s.at[srcb], hsb)

        def grp(g, _):
            q = pl.ds(g * 16, 16)
            mb[q] = a0b[q] * hsb[q]
            return 0
        lax.fori_loop(0, CH4 // 16, grp, 0)
        pltpu.sync_copy(mb, agg_s.at[dstb], add=True)
        return 0

    lax.fori_loop(0, EPW // CH4, chunk, 0)
    plsc.subcore_barrier()
    pltpu.sync_copy(agg_s.at[sl], out_h.at[c, sl])


def _layer0(src1, dst1, a0f, h0p):
    f = pl.kernel(
        _k4_body,
        out_type=jax.ShapeDtypeStruct((NC, NT), _F32),
        mesh=plsc.VectorSubcoreMesh(core_axis_name="c", subcore_axis_name="s"),
        compiler_params=pltpu.CompilerParams(needs_layout_passes=False),
        scratch_types=(
            [pltpu.VMEM_SHARED((NT,), _F32)] * 2
            + [pltpu.VMEM((CH4,), _I32)] * 2
            + [pltpu.VMEM((CH4,), _F32)] * 3
            + [pltpu.SemaphoreType.DMA]
        ),
    )
    return f(src1, dst1, a0f, h0p)


def _k6_body(src_h, dst_h, a1_h, b1_h, rx_h, ry_h, rz_h, h0_h, part_h, ws0_h,
             h0a_out, out_h,
             h0a_s, ag0, ag1, ag2, ag3,
             srcb, dstb, a1b, b1b, rxb, ryb, rzb, hsb,
             m0b, m1b, m2b, m3b, wsb, sem):
    c = lax.axis_index("c")
    s = lax.axis_index("s")
    w = s * NC + c
    sl = pl.ds(s * NTS, NTS)
    nts = pl.ds(0, NTS)
    _drain([pltpu.async_copy(part_h.at[0, sl], a1b.at[nts], sem),
            pltpu.async_copy(part_h.at[1, sl], b1b.at[nts], sem),
            pltpu.async_copy(h0_h.at[sl], rxb.at[nts], sem),
            pltpu.async_copy(ws0_h, wsb, sem)])

    def hloop(i, _):
        q = pl.ds(i * 16, 16)
        ryb[q] = a1b[q] + b1b[q] + rxb[q] * wsb[...]
        rzb[q] = jnp.zeros((16,), _F32)
        return 0
    lax.fori_loop(0, NTS // 16, hloop, 0)
    pltpu.sync_copy(ryb.at[nts], h0a_s.at[sl])

    @pl.when(c == 0)
    def _():
        pltpu.sync_copy(ryb.at[nts], h0a_out.at[sl])

    _drain([pltpu.async_copy(rzb.at[nts], ag0.at[sl], sem),
            pltpu.async_copy(rzb.at[nts], ag1.at[sl], sem),
            pltpu.async_copy(rzb.at[nts], ag2.at[sl], sem),
            pltpu.async_copy(rzb.at[nts], ag3.at[sl], sem)])
    plsc.subcore_barrier()

    def chunk(i, _):
        base = w * EPW + i * CH6
        ds = pl.ds(base, CH6)
        _drain([pltpu.async_copy(src_h.at[ds], srcb, sem),
                pltpu.async_copy(dst_h.at[ds], dstb, sem),
                pltpu.async_copy(a1_h.at[ds], a1b, sem),
                pltpu.async_copy(b1_h.at[ds], b1b, sem),
                pltpu.async_copy(rx_h.at[ds], rxb, sem),
                pltpu.async_copy(ry_h.at[ds], ryb, sem),
                pltpu.async_copy(rz_h.at[ds], rzb, sem)])
        pltpu.sync_copy(h0a_s.at[srcb], hsb)

        def grp(g, _):
            q = pl.ds(g * 16, 16)
            hs = hsb[q]
            t = b1b[q] * hs
            m0b[q] = a1b[q] * hs
            m1b[q] = t * rxb[q]
            m2b[q] = t * ryb[q]
            m3b[q] = t * rzb[q]
            return 0
        lax.fori_loop(0, CH6 // 16, grp, 0)
        _drain([pltpu.async_copy(m0b, ag0.at[dstb], sem, add=True),
                pltpu.async_copy(m1b, ag1.at[dstb], sem, add=True),
                pltpu.async_copy(m2b, ag2.at[dstb], sem, add=True),
                pltpu.async_copy(m3b, ag3.at[dstb], sem, add=True)])
        return 0

    lax.fori_loop(0, EPW // CH6, chunk, 0)
    plsc.subcore_barrier()
    _drain([pltpu.async_copy(ag0.at[sl], out_h.at[c, 0, sl], sem),
            pltpu.async_copy(ag1.at[sl], out_h.at[c, 1, sl], sem),
            pltpu.async_copy(ag2.at[sl], out_h.at[c, 2, sl], sem),
            pltpu.async_copy(ag3.at[sl], out_h.at[c, 3, sl], sem)])


def _layer1(src1, dst1, a1f, b1f, rx, ry, rz, h0p, part0, ws0v):
    f = pl.kernel(
        _k6_body,
        out_type=(jax.ShapeDtypeStruct((NT,), _F32),
                  jax.ShapeDtypeStruct((NC, 4, NT), _F32)),
        mesh=plsc.VectorSubcoreMesh(core_axis_name="c", subcore_axis_name="s"),
        compiler_params=pltpu.CompilerParams(needs_layout_passes=False),
        scratch_types=(
            [pltpu.VMEM_SHARED((NT,), _F32)] * 5
            + [pltpu.VMEM((CH6,), _I32)] * 2
            + [pltpu.VMEM((CH6,), _F32)] * 10
            + [pltpu.VMEM((16,), _F32)]
            + [pltpu.SemaphoreType.DMA]
        ),
    )
    return f(src1, dst1, a1f, b1f, rx, ry, rz, h0p, part0, ws0v)


def _k7_body(p0_ref, p1_ref, ha_ref, ws1_ref, o_ref):
    o = p0_ref[...] + p1_ref[...]
    row = lax.broadcasted_iota(_I32, o.shape, 0)
    ha = jnp.broadcast_to(ha_ref[...], o.shape)
    o_ref[...] = o + jnp.where(row == 0, ha * ws1_ref[0, 0], _f(0.0))


def _combine(p0, p1, h0a, ws1):
    bt = 12544
    blk = pl.BlockSpec((4, bt), lambda i: (0, i))
    return pl.pallas_call(
        _k7_body,
        grid=(NT // bt,),
        in_specs=[blk, blk,
                  pl.BlockSpec((1, bt), lambda i: (0, i)),
                  pl.BlockSpec(memory_space=pltpu.SMEM)],
        out_specs=blk,
        out_shape=jax.ShapeDtypeStruct((4, NT), _F32),
    )(p0, p1, h0a.reshape(1, NT), ws1)


def kernel(x, pos, edge_index, edge_attr, W1, b1, W2, b2,
           Wr1_0, br1_0, Wr2_0, br2_0, ws_0,
           Wr1_1, br1_1, Wr2_1, br2_1, ws_1):
    src1 = jnp.pad(edge_index[0].astype(_I32), (0, EPAD - E))
    dst1 = jnp.pad(edge_index[1].astype(_I32), (0, EPAD - E), constant_values=N)
    ea0 = jnp.pad(edge_attr[:, 0], (0, EPAD - E)).reshape(EROWS, LAN)
    ea1 = jnp.pad(edge_attr[:, 1], (0, EPAD - E)).reshape(EROWS, LAN)
    xb = lax.bitcast_convert_type(pos[:, 0].astype(jnp.bfloat16), jnp.uint16)
    yb = lax.bitcast_convert_type(pos[:, 1].astype(jnp.bfloat16), jnp.uint16)
    packed = xb.astype(jnp.uint32) | (yb.astype(jnp.uint32) << 16)
    pxy = jnp.pad(lax.bitcast_convert_type(packed, _F32), (0, NT - N))
    posz = jnp.pad(pos[:, 2], (0, NT - N))

    h0 = _node_mlp(x, W1, b1, W2, b2)
    h0p = jnp.pad(h0[:, 0], (0, NT - N))
    rx, ry, rz, r1 = _edge_r(src1, dst1, pxy, posz)
    a0, a1, b1r = _radial(ea0, ea1, r1.reshape(EROWS, LAN),
                          Wr1_0, br1_0, Wr2_0, br2_0,
                          Wr1_1, br1_1, Wr2_1, br2_1)
    part0 = _layer0(src1, dst1, a0.reshape(EPAD), h0p)
    ws0v = jnp.broadcast_to(ws_0.reshape(1), (16,))
    h0a, part4 = _layer1(src1, dst1, a1.reshape(EPAD), b1r.reshape(EPAD),
                         rx, ry, rz, h0p, part0, ws0v)
    out4 = _combine(part4[0], part4[1], h0a, ws_1)
    return out4[:, :N].T


# speedup vs baseline: 89.2985x; 1.0306x over previous
import numpy as np

import jax
import jax.numpy as jnp
from jax import lax
from jax.experimental import pallas as pl
from jax.experimental.pallas import tpu as pltpu
from jax.experimental.pallas import tpu_sc as plsc

N = 100000
E = 3200000
D_IN = 67
H = 32

LAN = 128
NC = 2
NS = 16
NW = NC * NS
EPW = 100352
EROWS = 25088
EPAD = NW * EPW
NT = 100352
NTS = NT // NS

CH2 = 7168
CH4 = 14336
CH6 = 7168

_F32 = jnp.float32
_I32 = jnp.int32
_f = np.float32
_MAGIC = np.int32(0x5F3759DF)


def _newton_r(dx, dy, dz):
    rsq = dx * dx + dy * dy + dz * dz
    bits = plsc.bitcast(rsq, _I32)
    y = plsc.bitcast(_MAGIC - lax.shift_right_logical(bits, 1), _F32)
    hr = rsq * _f(0.5)
    for _ in range(3):
        y = y * (_f(1.5) - hr * y * y)
    return rsq * y


def _drain(descs):
    for d in descs:
        d.wait()


def _k1_body(x_ref, w1_ref, b1_ref, w2_ref, b2_ref, o_ref):
    xb = x_ref[...].astype(jnp.bfloat16)
    w1 = w1_ref[...].astype(jnp.bfloat16)
    l0 = jnp.dot(xb, w1, preferred_element_type=_F32)
    l0 = l0 + b1_ref[...]
    l0 = jnp.where(l0 > 0, l0, jnp.exp(l0) - _f(1.0))
    h = jnp.dot(l0.astype(jnp.bfloat16), w2_ref[...].astype(jnp.bfloat16),
                preferred_element_type=_F32) + b2_ref[0, 0]
    o_ref[...] = h


def _node_mlp(x, W1, b1, W2, b2):
    nb = 1000
    return pl.pallas_call(
        _k1_body,
        grid=(N // nb,),
        in_specs=[
            pl.BlockSpec((nb, D_IN), lambda i: (i, 0)),
            pl.BlockSpec((D_IN, D_IN), lambda i: (0, 0)),
            pl.BlockSpec((1, D_IN), lambda i: (0, 0)),
            pl.BlockSpec((D_IN, 1), lambda i: (0, 0)),
            pl.BlockSpec(memory_space=pltpu.SMEM),
        ],
        out_specs=pl.BlockSpec((nb, 1), lambda i: (i, 0)),
        out_shape=jax.ShapeDtypeStruct((N, 1), _F32),
    )(x, W1, b1.reshape(1, D_IN), W2, b2.reshape(1, 1))


def _k2_body(src_h, dst_h, pxy_h, pz_h,
             rx_h, ry_h, rz_h, r_h,
             pxy_s, pz_s,
             srcb, dstb, gxys, gzs, gxyd, gzd, gys, rb, sem):
    c = lax.axis_index("c")
    s = lax.axis_index("s")
    w = s * NC + c
    sl = pl.ds(s * NTS, NTS)
    _drain([pltpu.async_copy(pxy_h.at[sl], pxy_s.at[sl], sem),
            pltpu.async_copy(pz_h.at[sl], pz_s.at[sl], sem)])
    plsc.subcore_barrier()

    def chunk(i, _):
        base = w * EPW + i * CH2
        ds = pl.ds(base, CH2)
        _drain([pltpu.async_copy(src_h.at[ds], srcb, sem),
                pltpu.async_copy(dst_h.at[ds], dstb, sem)])
        _drain([pltpu.async_copy(pxy_s.at[srcb], gxys, sem),
                pltpu.async_copy(pz_s.at[srcb], gzs, sem),
                pltpu.async_copy(pxy_s.at[dstb], gxyd, sem),
                pltpu.async_copy(pz_s.at[dstb], gzd, sem)])

        def grp(g, _):
            q = pl.ds(g * 16, 16)
            dxy = plsc.bitcast(gxys[q], jnp.bfloat16) - plsc.bitcast(gxyd[q], jnp.bfloat16)
            dx, dy = plsc.unpack(dxy, format=plsc.PackFormat.INTERLEAVED)
            dz = gzs[q] - gzd[q]
            r = _newton_r(dx, dy, dz)
            qq = _f(1.0) / (r + _f(1e-8))
            rb[q] = r
            gxys[q] = dx * qq
            gys[q] = dy * qq
            gzs[q] = dz * qq
            return 0
        lax.fori_loop(0, CH2 // 16, grp, 0)
        _drain([pltpu.async_copy(rb, r_h.at[ds], sem),
                pltpu.async_copy(gxys, rx_h.at[ds], sem),
                pltpu.async_copy(gys, ry_h.at[ds], sem),
                pltpu.async_copy(gzs, rz_h.at[ds], sem)])
        return 0

    lax.fori_loop(0, EPW // CH2, chunk, 0)


def _edge_r(src1, dst1, pxy, posz):
    f = pl.kernel(
        _k2_body,
        out_type=(jax.ShapeDtypeStruct((EPAD,), _F32),) * 4,
        mesh=plsc.VectorSubcoreMesh(core_axis_name="c", subcore_axis_name="s"),
        compiler_params=pltpu.CompilerParams(needs_layout_passes=False),
        scratch_types=(
            [pltpu.VMEM_SHARED((NT,), _F32)] * 2
            + [pltpu.VMEM((CH2,), _I32)] * 2
            + [pltpu.VMEM((CH2,), _F32)] * 6
            + [pltpu.SemaphoreType.DMA]
        ),
    )
    return f(src1, dst1, pxy, posz)


def _k3_body(e0_ref, e1_ref, r_ref, w10, b10, w20, b20, w11, b11, w21, b21,
             a0_ref, a1_ref, b1_ref):
    def b16(v):
        return v.astype(jnp.bfloat16).astype(_F32)

    e0 = b16(e0_ref[...])
    e1 = b16(e1_ref[...])
    rr = b16(r_ref[...])
    acc0 = jnp.zeros(e0.shape, _F32)
    acc1 = jnp.zeros(e0.shape, _F32)
    accb = jnp.zeros(e0.shape, _F32)
    for j in range(H):
        h0 = e0 * b16(w10[0, j]) + e1 * b16(w10[1, j]) + rr * b16(w10[2, j])
        h0 = jnp.maximum(h0 + b10[0, j], _f(0.0))
        acc0 = acc0 + h0 * b16(w20[j, 0])
        h1 = e0 * b16(w11[0, j]) + e1 * b16(w11[1, j]) + rr * b16(w11[2, j])
        h1 = jnp.maximum(h1 + b11[0, j], _f(0.0))
        acc1 = acc1 + h1 * b16(w21[j, 0])
        accb = accb + h1 * b16(w21[j, 1])
    a0_ref[...] = acc0 + b20[0, 0]
    a1_ref[...] = acc1 + b21[0, 0]
    b1_ref[...] = accb + b21[0, 1]


def _radial(ea0, ea1, r2, Wr1_0, br1_0, Wr2_0, br2_0, Wr1_1, br1_1, Wr2_1, br2_1):
    rb = 512
    smem = pl.BlockSpec(memory_space=pltpu.SMEM)
    blk = pl.BlockSpec((rb, LAN), lambda i: (i, 0))
    return pl.pallas_call(
        _k3_body,
        grid=(EROWS // rb,),
        in_specs=[blk, blk, blk] + [smem] * 8,
        out_specs=[blk, blk, blk],
        out_shape=[jax.ShapeDtypeStruct((EROWS, LAN), _F32)] * 3,
    )(ea0, ea1, r2,
      Wr1_0, br1_0.reshape(1, H), Wr2_0, br2_0.reshape(1, 2),
      Wr1_1, br1_1.reshape(1, H), Wr2_1, br2_1.reshape(1, 2))


def _k4_body(src_h, dst_h, a0_h, h0_h, out_h,
             h0_s, agg_s,
             srcb, dstb, a0b, hsb, mb, sem):
    c = lax.axis_index("c")
    s = lax.axis_index("s")
    w = s * NC + c
    sl = pl.ds(s * NTS, NTS)
    pltpu.sync_copy(h0_h.at[sl], h0_s.at[sl])

    def zloop(i, _):
        mb[pl.ds(i * 16, 16)] = jnp.zeros((16,), _F32)
        return 0
    lax.fori_loop(0, NTS // 16, zloop, 0)
    pltpu.sync_copy(mb.at[pl.ds(0, NTS)], agg_s.at[sl])
    plsc.subcore_barrier()

    def chunk(i, _):
        base = w * EPW + i * CH4
        ds = pl.ds(base, CH4)
        _drain([pltpu.async_copy(src_h.at[ds], srcb, sem),
                pltpu.async_copy(dst_h.at[ds], dstb, sem),
                pltpu.async_copy(a0_h.at[ds], a0b, sem)])
        pltpu.sync_copy(h0_s.at[srcb], hsb)

        def grp(g, _):
            q = pl.ds(g * 16, 16)
            mb[q] = a0b[q] * hsb[q]
            return 0
        lax.fori_loop(0, CH4 // 16, grp, 0)
        pltpu.sync_copy(mb, agg_s.at[dstb], add=True)
        return 0

    lax.fori_loop(0, EPW // CH4, chunk, 0)
    plsc.subcore_barrier()
    pltpu.sync_copy(agg_s.at[sl], out_h.at[c, sl])


def _layer0(src1, dst1, a0f, h0p):
    f = pl.kernel(
        _k4_body,
        out_type=jax.ShapeDtypeStruct((NC, NT), _F32),
        mesh=plsc.VectorSubcoreMesh(core_axis_name="c", subcore_axis_name="s"),
        compiler_params=pltpu.CompilerParams(needs_layout_passes=False),
        scratch_types=(
            [pltpu.VMEM_SHARED((NT,), _F32)] * 2
            + [pltpu.VMEM((CH4,), _I32)] * 2
            + [pltpu.VMEM((CH4,), _F32)] * 3
            + [pltpu.SemaphoreType.DMA]
        ),
    )
    return f(src1, dst1, a0f, h0p)


def _k6_body(src_h, dst_h, a1_h, b1_h, rx_h, ry_h, rz_h, h0_h, part_h, ws0_h,
             h0a_out, out_h,
             h0a_s, ag0, ag1, ag2, ag3,
             srcb, dstb, a1b, b1b, rxb, ryb, rzb, hsb,
             m0b, m1b, m2b, m3b, wsb, sem):
    c = lax.axis_index("c")
    s = lax.axis_index("s")
    w = s * NC + c
    sl = pl.ds(s * NTS, NTS)
    nts = pl.ds(0, NTS)
    _drain([pltpu.async_copy(part_h.at[0, sl], a1b.at[nts], sem),
            pltpu.async_copy(part_h.at[1, sl], b1b.at[nts], sem),
            pltpu.async_copy(h0_h.at[sl], rxb.at[nts], sem),
            pltpu.async_copy(ws0_h, wsb, sem)])

    def hloop(i, _):
        q = pl.ds(i * 16, 16)
        ryb[q] = a1b[q] + b1b[q] + rxb[q] * wsb[...]
        rzb[q] = jnp.zeros((16,), _F32)
        return 0
    lax.fori_loop(0, NTS // 16, hloop, 0)
    pltpu.sync_copy(ryb.at[nts], h0a_s.at[sl])

    @pl.when(c == 0)
    def _():
        pltpu.sync_copy(ryb.at[nts], h0a_out.at[sl])

    _drain([pltpu.async_copy(rzb.at[nts], ag0.at[sl], sem),
            pltpu.async_copy(rzb.at[nts], ag1.at[sl], sem),
            pltpu.async_copy(rzb.at[nts], ag2.at[sl], sem),
            pltpu.async_copy(rzb.at[nts], ag3.at[sl], sem)])
    plsc.subcore_barrier()

    def chunk(i, _):
        base = w * EPW + i * CH6
        ds = pl.ds(base, CH6)
        _drain([pltpu.async_copy(src_h.at[ds], srcb, sem),
                pltpu.async_copy(dst_h.at[ds], dstb, sem),
                pltpu.async_copy(a1_h.at[ds], a1b, sem),
                pltpu.async_copy(b1_h.at[ds], b1b, sem),
                pltpu.async_copy(rx_h.at[ds], rxb, sem),
                pltpu.async_copy(ry_h.at[ds], ryb, sem),
                pltpu.async_copy(rz_h.at[ds], rzb, sem)])
        pltpu.sync_copy(h0a_s.at[srcb], hsb)

        def grp(g, _):
            q = pl.ds(g * 16, 16)
            hs = hsb[q]
            t = b1b[q] * hs
            m0b[q] = a1b[q] * hs
            m1b[q] = t * rxb[q]
            m2b[q] = t * ryb[q]
            m3b[q] = t * rzb[q]
            return 0
        lax.fori_loop(0, CH6 // 16, grp, 0)
        _drain([pltpu.async_copy(m0b, ag0.at[dstb], sem, add=True),
                pltpu.async_copy(m1b, ag1.at[dstb], sem, add=True),
                pltpu.async_copy(m2b, ag2.at[dstb], sem, add=True),
                pltpu.async_copy(m3b, ag3.at[dstb], sem, add=True)])
        return 0

    lax.fori_loop(0, EPW // CH6, chunk, 0)
    plsc.subcore_barrier()
    _drain([pltpu.async_copy(ag0.at[sl], out_h.at[c, 0, sl], sem),
            pltpu.async_copy(ag1.at[sl], out_h.at[c, 1, sl], sem),
            pltpu.async_copy(ag2.at[sl], out_h.at[c, 2, sl], sem),
            pltpu.async_copy(ag3.at[sl], out_h.at[c, 3, sl], sem)])


def _layer1(src1, dst1, a1f, b1f, rx, ry, rz, h0p, part0, ws0v):
    f = pl.kernel(
        _k6_body,
        out_type=(jax.ShapeDtypeStruct((NT,), _F32),
                  jax.ShapeDtypeStruct((NC, 4, NT), _F32)),
        mesh=plsc.VectorSubcoreMesh(core_axis_name="c", subcore_axis_name="s"),
        compiler_params=pltpu.CompilerParams(needs_layout_passes=False),
        scratch_types=(
            [pltpu.VMEM_SHARED((NT,), _F32)] * 5
            + [pltpu.VMEM((CH6,), _I32)] * 2
            + [pltpu.VMEM((CH6,), _F32)] * 10
            + [pltpu.VMEM((16,), _F32)]
            + [pltpu.SemaphoreType.DMA]
        ),
    )
    return f(src1, dst1, a1f, b1f, rx, ry, rz, h0p, part0, ws0v)


def _k7_body(p0_ref, p1_ref, ha_ref, ws1_ref, o_ref):
    o = p0_ref[...] + p1_ref[...]
    row = lax.broadcasted_iota(_I32, o.shape, 0)
    ha = jnp.broadcast_to(ha_ref[...], o.shape)
    o_ref[...] = o + jnp.where(row == 0, ha * ws1_ref[0, 0], _f(0.0))


def _combine(p0, p1, h0a, ws1):
    bt = 12544
    blk = pl.BlockSpec((4, bt), lambda i: (0, i))
    return pl.pallas_call(
        _k7_body,
        grid=(NT // bt,),
        in_specs=[blk, blk,
                  pl.BlockSpec((1, bt), lambda i: (0, i)),
                  pl.BlockSpec(memory_space=pltpu.SMEM)],
        out_specs=blk,
        out_shape=jax.ShapeDtypeStruct((4, NT), _F32),
    )(p0, p1, h0a.reshape(1, NT), ws1)


def kernel(x, pos, edge_index, edge_attr, W1, b1, W2, b2,
           Wr1_0, br1_0, Wr2_0, br2_0, ws_0,
           Wr1_1, br1_1, Wr2_1, br2_1, ws_1):
    src1 = jnp.pad(edge_index[0].astype(_I32), (0, EPAD - E))
    dst1 = jnp.pad(edge_index[1].astype(_I32), (0, EPAD - E), constant_values=N)
    ea0 = jnp.pad(edge_attr[:, 0], (0, EPAD - E)).reshape(EROWS, LAN)
    ea1 = jnp.pad(edge_attr[:, 1], (0, EPAD - E)).reshape(EROWS, LAN)
    xb = lax.bitcast_convert_type(pos[:, 0].astype(jnp.bfloat16), jnp.uint16)
    yb = lax.bitcast_convert_type(pos[:, 1].astype(jnp.bfloat16), jnp.uint16)
    packed = xb.astype(jnp.uint32) | (yb.astype(jnp.uint32) << 16)
    pxy = jnp.pad(lax.bitcast_convert_type(packed, _F32), (0, NT - N))
    posz = jnp.pad(pos[:, 2], (0, NT - N))

    h0 = _node_mlp(x, W1, b1, W2, b2)
    h0p = jnp.pad(h0[:, 0], (0, NT - N))
    rx, ry, rz, r1 = _edge_r(src1, dst1, pxy, posz)
    a0, a1, b1r = _radial(ea0, ea1, r1.reshape(EROWS, LAN),
                          Wr1_0, br1_0, Wr2_0, br2_0,
                          Wr1_1, br1_1, Wr2_1, br2_1)
    part0 = _layer0(src1, dst1, a0.reshape(EPAD), h0p)
    ws0v = jnp.broadcast_to(ws_0.reshape(1), (16,))
    h0a, part4 = _layer1(src1, dst1, a1.reshape(EPAD), b1r.reshape(EPAD),
                         rx, ry, rz, h0p, part0, ws0v)
    out4 = _combine(part4[0], part4[1], h0a, ws_1)
    return out4[:, :N].T


# speedup vs baseline: 104.8867x; 1.1746x over previous
import numpy as np

import jax
import jax.numpy as jnp
from jax import lax
from jax.experimental import pallas as pl
from jax.experimental.pallas import tpu as pltpu
from jax.experimental.pallas import tpu_sc as plsc

N = 100000
E = 3200000
D_IN = 67
H = 32

LAN = 128
NC = 2
NS = 16
NW = NC * NS
EPW = 100352
EROWS = 25088
EPAD = NW * EPW
NT = 100352
NTS = NT // NS

CH2 = 7168
CH4 = 14336
CH6 = 7168

_F32 = jnp.float32
_I32 = jnp.int32
_f = np.float32
_MAGIC = np.int32(0x5F3759DF)


def _newton_r(dx, dy, dz):
    rsq = dx * dx + dy * dy + dz * dz
    bits = plsc.bitcast(rsq, _I32)
    y = plsc.bitcast(_MAGIC - lax.shift_right_logical(bits, 1), _F32)
    hr = rsq * _f(0.5)
    for _ in range(3):
        y = y * (_f(1.5) - hr * y * y)
    return rsq * y


def _drain(descs):
    for d in descs:
        d.wait()


def _k1_body(x_ref, w1_ref, b1_ref, w2_ref, b2_ref, o_ref):
    xb = x_ref[...].astype(jnp.bfloat16)
    w1 = w1_ref[...].astype(jnp.bfloat16)
    l0 = jnp.dot(xb, w1, preferred_element_type=_F32)
    l0 = l0 + b1_ref[...]
    l0 = jnp.where(l0 > 0, l0, jnp.exp(l0) - _f(1.0))
    h = jnp.dot(l0.astype(jnp.bfloat16), w2_ref[...].astype(jnp.bfloat16),
                preferred_element_type=_F32) + b2_ref[0, 0]
    o_ref[...] = h


def _node_mlp(x, W1, b1, W2, b2):
    nb = 1000
    return pl.pallas_call(
        _k1_body,
        grid=(N // nb,),
        in_specs=[
            pl.BlockSpec((nb, D_IN), lambda i: (i, 0)),
            pl.BlockSpec((D_IN, D_IN), lambda i: (0, 0)),
            pl.BlockSpec((1, D_IN), lambda i: (0, 0)),
            pl.BlockSpec((D_IN, 1), lambda i: (0, 0)),
            pl.BlockSpec(memory_space=pltpu.SMEM),
        ],
        out_specs=pl.BlockSpec((nb, 1), lambda i: (i, 0)),
        out_shape=jax.ShapeDtypeStruct((N, 1), _F32),
    )(x, W1, b1.reshape(1, D_IN), W2, b2.reshape(1, 1))


def _k2_chunk_compute(gxys, gxyd, gzs, gzd, rxo, ryo, rzo, ro):
    def grp(g, _):
        q = pl.ds(g * 16, 16)
        dxy = plsc.bitcast(gxys[q], jnp.bfloat16) - plsc.bitcast(gxyd[q], jnp.bfloat16)
        dx, dy = plsc.unpack(dxy, format=plsc.PackFormat.INTERLEAVED)
        dz = gzs[q] - gzd[q]
        r = _newton_r(dx, dy, dz)
        qq = _f(1.0) / (r + _f(1e-8))
        ro[q] = r
        rxo[q] = dx * qq
        ryo[q] = dy * qq
        rzo[q] = dz * qq
        return 0
    lax.fori_loop(0, CH2 // 16, grp, 0)


def _k2_body(src_h, dst_h, pxy_h, pz_h,
             rx_h, ry_h, rz_h, r_h,
             pxy_s, pz_s,
             srcA, dstA, xyA, zA, xydA, zdA,
             srcB, dstB, xyB, zB, xydB, zdB,
             rxo, ryo, rzo, ro,
             lsem, gsem, wsem):
    c = lax.axis_index("c")
    s = lax.axis_index("s")
    w = s * NC + c
    sl = pl.ds(s * NTS, NTS)
    _drain([pltpu.async_copy(pxy_h.at[sl], pxy_s.at[sl], lsem),
            pltpu.async_copy(pz_h.at[sl], pz_s.at[sl], lsem)])
    plsc.subcore_barrier()

    sets = [(srcA, dstA, xyA, zA, xydA, zdA), (srcB, dstB, xyB, zB, xydB, zdB)]
    nch = EPW // CH2

    def lin(j):
        sb, db = sets[j % 2][0], sets[j % 2][1]
        ds = pl.ds(w * EPW + j * CH2, CH2)
        return [pltpu.async_copy(src_h.at[ds], sb, lsem),
                pltpu.async_copy(dst_h.at[ds], db, lsem)]

    def gat(j):
        sb, db, xy, z, xyd, zd = sets[j % 2]
        return [pltpu.async_copy(pxy_s.at[sb], xy, gsem),
                pltpu.async_copy(pz_s.at[sb], z, gsem),
                pltpu.async_copy(pxy_s.at[db], xyd, gsem),
                pltpu.async_copy(pz_s.at[db], zd, gsem)]

    def wr(j):
        ds = pl.ds(w * EPW + j * CH2, CH2)
        return [pltpu.async_copy(ro, r_h.at[ds], wsem),
                pltpu.async_copy(rxo, rx_h.at[ds], wsem),
                pltpu.async_copy(ryo, ry_h.at[ds], wsem),
                pltpu.async_copy(rzo, rz_h.at[ds], wsem)]

    _drain(lin(0))
    g_cur = gat(0)
    l_nxt = lin(1)
    w_prev = None
    for j in range(nch):
        _drain(g_cur)
        if j + 1 < nch:
            _drain(l_nxt)
            g_cur = gat(j + 1)
        if j + 2 < nch:
            l_nxt = lin(j + 2)
        if w_prev is not None:
            _drain(w_prev)
        _, _, xy, z, xyd, zd = sets[j % 2]
        _k2_chunk_compute(xy, xyd, z, zd, rxo, ryo, rzo, ro)
        w_prev = wr(j)
    _drain(w_prev)


def _edge_r(src1, dst1, pxy, posz):
    f = pl.kernel(
        _k2_body,
        out_type=(jax.ShapeDtypeStruct((EPAD,), _F32),) * 4,
        mesh=plsc.VectorSubcoreMesh(core_axis_name="c", subcore_axis_name="s"),
        compiler_params=pltpu.CompilerParams(needs_layout_passes=False),
        scratch_types=(
            [pltpu.VMEM_SHARED((NT,), _F32)] * 2
            + ([pltpu.VMEM((CH2,), _I32)] * 2
               + [pltpu.VMEM((CH2,), _F32)] * 4) * 2
            + [pltpu.VMEM((CH2,), _F32)] * 4
            + [pltpu.SemaphoreType.DMA] * 3
        ),
    )
    return f(src1, dst1, pxy, posz)


def _k3_body(e0_ref, e1_ref, r_ref, w10, b10, w20, b20, w11, b11, w21, b21,
             a0_ref, a1_ref, b1_ref):
    def b16(v):
        return v.astype(jnp.bfloat16).astype(_F32)

    e0 = b16(e0_ref[...])
    e1 = b16(e1_ref[...])
    rr = b16(r_ref[...])
    acc0 = jnp.zeros(e0.shape, _F32)
    acc1 = jnp.zeros(e0.shape, _F32)
    accb = jnp.zeros(e0.shape, _F32)
    for j in range(H):
        h0 = e0 * b16(w10[0, j]) + e1 * b16(w10[1, j]) + rr * b16(w10[2, j])
        h0 = jnp.maximum(h0 + b10[0, j], _f(0.0))
        acc0 = acc0 + h0 * b16(w20[j, 0])
        h1 = e0 * b16(w11[0, j]) + e1 * b16(w11[1, j]) + rr * b16(w11[2, j])
        h1 = jnp.maximum(h1 + b11[0, j], _f(0.0))
        acc1 = acc1 + h1 * b16(w21[j, 0])
        accb = accb + h1 * b16(w21[j, 1])
    a0_ref[...] = acc0 + b20[0, 0]
    a1_ref[...] = acc1 + b21[0, 0]
    b1_ref[...] = accb + b21[0, 1]


def _radial(ea0, ea1, r2, Wr1_0, br1_0, Wr2_0, br2_0, Wr1_1, br1_1, Wr2_1, br2_1):
    rb = 512
    smem = pl.BlockSpec(memory_space=pltpu.SMEM)
    blk = pl.BlockSpec((rb, LAN), lambda i: (i, 0))
    return pl.pallas_call(
        _k3_body,
        grid=(EROWS // rb,),
        in_specs=[blk, blk, blk] + [smem] * 8,
        out_specs=[blk, blk, blk],
        out_shape=[jax.ShapeDtypeStruct((EROWS, LAN), _F32)] * 3,
    )(ea0, ea1, r2,
      Wr1_0, br1_0.reshape(1, H), Wr2_0, br2_0.reshape(1, 2),
      Wr1_1, br1_1.reshape(1, H), Wr2_1, br2_1.reshape(1, 2))


def _k4_body(src_h, dst_h, a0_h, h0_h, out_h,
             h0_s, agg_s,
             srcb, dstb, a0b, hsb, mb, sem):
    c = lax.axis_index("c")
    s = lax.axis_index("s")
    w = s * NC + c
    sl = pl.ds(s * NTS, NTS)
    pltpu.sync_copy(h0_h.at[sl], h0_s.at[sl])

    def zloop(i, _):
        mb[pl.ds(i * 16, 16)] = jnp.zeros((16,), _F32)
        return 0
    lax.fori_loop(0, NTS // 16, zloop, 0)
    pltpu.sync_copy(mb.at[pl.ds(0, NTS)], agg_s.at[sl])
    plsc.subcore_barrier()

    def chunk(i, _):
        base = w * EPW + i * CH4
        ds = pl.ds(base, CH4)
        _drain([pltpu.async_copy(src_h.at[ds], srcb, sem),
                pltpu.async_copy(dst_h.at[ds], dstb, sem),
                pltpu.async_copy(a0_h.at[ds], a0b, sem)])
        pltpu.sync_copy(h0_s.at[srcb], hsb)

        def grp(g, _):
            q = pl.ds(g * 16, 16)
            mb[q] = a0b[q] * hsb[q]
            return 0
        lax.fori_loop(0, CH4 // 16, grp, 0)
        pltpu.sync_copy(mb, agg_s.at[dstb], add=True)
        return 0

    lax.fori_loop(0, EPW // CH4, chunk, 0)
    plsc.subcore_barrier()
    pltpu.sync_copy(agg_s.at[sl], out_h.at[c, sl])


def _layer0(src1, dst1, a0f, h0p):
    f = pl.kernel(
        _k4_body,
        out_type=jax.ShapeDtypeStruct((NC, NT), _F32),
        mesh=plsc.VectorSubcoreMesh(core_axis_name="c", subcore_axis_name="s"),
        compiler_params=pltpu.CompilerParams(needs_layout_passes=False),
        scratch_types=(
            [pltpu.VMEM_SHARED((NT,), _F32)] * 2
            + [pltpu.VMEM((CH4,), _I32)] * 2
            + [pltpu.VMEM((CH4,), _F32)] * 3
            + [pltpu.SemaphoreType.DMA]
        ),
    )
    return f(src1, dst1, a0f, h0p)


def _k6_body(src_h, dst_h, a1_h, b1_h, rx_h, ry_h, rz_h, h0_h, part_h, ws0_h,
             h0a_out, out_h,
             h0a_s, ag0, ag1, ag2, ag3,
             srcb, dstb, a1b, b1b, rxb, ryb, rzb, hsb,
             m0b, m1b, m2b, m3b, wsb, sem):
    c = lax.axis_index("c")
    s = lax.axis_index("s")
    w = s * NC + c
    sl = pl.ds(s * NTS, NTS)
    nts = pl.ds(0, NTS)
    _drain([pltpu.async_copy(part_h.at[0, sl], a1b.at[nts], sem),
            pltpu.async_copy(part_h.at[1, sl], b1b.at[nts], sem),
            pltpu.async_copy(h0_h.at[sl], rxb.at[nts], sem),
            pltpu.async_copy(ws0_h, wsb, sem)])

    def hloop(i, _):
        q = pl.ds(i * 16, 16)
        ryb[q] = a1b[q] + b1b[q] + rxb[q] * wsb[...]
        rzb[q] = jnp.zeros((16,), _F32)
        return 0
    lax.fori_loop(0, NTS // 16, hloop, 0)
    pltpu.sync_copy(ryb.at[nts], h0a_s.at[sl])

    @pl.when(c == 0)
    def _():
        pltpu.sync_copy(ryb.at[nts], h0a_out.at[sl])

    _drain([pltpu.async_copy(rzb.at[nts], ag0.at[sl], sem),
            pltpu.async_copy(rzb.at[nts], ag1.at[sl], sem),
            pltpu.async_copy(rzb.at[nts], ag2.at[sl], sem),
            pltpu.async_copy(rzb.at[nts], ag3.at[sl], sem)])
    plsc.subcore_barrier()

    def chunk(i, _):
        base = w * EPW + i * CH6
        ds = pl.ds(base, CH6)
        _drain([pltpu.async_copy(src_h.at[ds], srcb, sem),
                pltpu.async_copy(dst_h.at[ds], dstb, sem),
                pltpu.async_copy(a1_h.at[ds], a1b, sem),
                pltpu.async_copy(b1_h.at[ds], b1b, sem),
                pltpu.async_copy(rx_h.at[ds], rxb, sem),
                pltpu.async_copy(ry_h.at[ds], ryb, sem),
                pltpu.async_copy(rz_h.at[ds], rzb, sem)])
        pltpu.sync_copy(h0a_s.at[srcb], hsb)

        def grp(g, _):
            q = pl.ds(g * 16, 16)
            hs = hsb[q]
            t = b1b[q] * hs
            m0b[q] = a1b[q] * hs
            m1b[q] = t * rxb[q]
            m2b[q] = t * ryb[q]
            m3b[q] = t * rzb[q]
            return 0
        lax.fori_loop(0, CH6 // 16, grp, 0)
        _drain([pltpu.async_copy(m0b, ag0.at[dstb], sem, add=True),
                pltpu.async_copy(m1b, ag1.at[dstb], sem, add=True),
                pltpu.async_copy(m2b, ag2.at[dstb], sem, add=True),
                pltpu.async_copy(m3b, ag3.at[dstb], sem, add=True)])
        return 0

    lax.fori_loop(0, EPW // CH6, chunk, 0)
    plsc.subcore_barrier()
    _drain([pltpu.async_copy(ag0.at[sl], out_h.at[c, 0, sl], sem),
            pltpu.async_copy(ag1.at[sl], out_h.at[c, 1, sl], sem),
            pltpu.async_copy(ag2.at[sl], out_h.at[c, 2, sl], sem),
            pltpu.async_copy(ag3.at[sl], out_h.at[c, 3, sl], sem)])


def _layer1(src1, dst1, a1f, b1f, rx, ry, rz, h0p, part0, ws0v):
    f = pl.kernel(
        _k6_body,
        out_type=(jax.ShapeDtypeStruct((NT,), _F32),
                  jax.ShapeDtypeStruct((NC, 4, NT), _F32)),
        mesh=plsc.VectorSubcoreMesh(core_axis_name="c", subcore_axis_name="s"),
        compiler_params=pltpu.CompilerParams(needs_layout_passes=False),
        scratch_types=(
            [pltpu.VMEM_SHARED((NT,), _F32)] * 5
            + [pltpu.VMEM((CH6,), _I32)] * 2
            + [pltpu.VMEM((CH6,), _F32)] * 10
            + [pltpu.VMEM((16,), _F32)]
            + [pltpu.SemaphoreType.DMA]
        ),
    )
    return f(src1, dst1, a1f, b1f, rx, ry, rz, h0p, part0, ws0v)


def _k7_body(p0_ref, p1_ref, ha_ref, ws1_ref, o_ref):
    o = p0_ref[...] + p1_ref[...]
    row = lax.broadcasted_iota(_I32, o.shape, 0)
    ha = jnp.broadcast_to(ha_ref[...], o.shape)
    o_ref[...] = o + jnp.where(row == 0, ha * ws1_ref[0, 0], _f(0.0))


def _combine(p0, p1, h0a, ws1):
    bt = 12544
    blk = pl.BlockSpec((4, bt), lambda i: (0, i))
    return pl.pallas_call(
        _k7_body,
        grid=(NT // bt,),
        in_specs=[blk, blk,
                  pl.BlockSpec((1, bt), lambda i: (0, i)),
                  pl.BlockSpec(memory_space=pltpu.SMEM)],
        out_specs=blk,
        out_shape=jax.ShapeDtypeStruct((4, NT), _F32),
    )(p0, p1, h0a.reshape(1, NT), ws1)


def kernel(x, pos, edge_index, edge_attr, W1, b1, W2, b2,
           Wr1_0, br1_0, Wr2_0, br2_0, ws_0,
           Wr1_1, br1_1, Wr2_1, br2_1, ws_1):
    src1 = jnp.pad(edge_index[0].astype(_I32), (0, EPAD - E))
    dst1 = jnp.pad(edge_index[1].astype(_I32), (0, EPAD - E), constant_values=N)
    ea0 = jnp.pad(edge_attr[:, 0], (0, EPAD - E)).reshape(EROWS, LAN)
    ea1 = jnp.pad(edge_attr[:, 1], (0, EPAD - E)).reshape(EROWS, LAN)
    xb = lax.bitcast_convert_type(pos[:, 0].astype(jnp.bfloat16), jnp.uint16)
    yb = lax.bitcast_convert_type(pos[:, 1].astype(jnp.bfloat16), jnp.uint16)
    packed = xb.astype(jnp.uint32) | (yb.astype(jnp.uint32) << 16)
    pxy = jnp.pad(lax.bitcast_convert_type(packed, _F32), (0, NT - N))
    posz = jnp.pad(pos[:, 2], (0, NT - N))

    h0 = _node_mlp(x, W1, b1, W2, b2)
    h0p = jnp.pad(h0[:, 0], (0, NT - N))
    rx, ry, rz, r1 = _edge_r(src1, dst1, pxy, posz)
    a0, a1, b1r = _radial(ea0, ea1, r1.reshape(EROWS, LAN),
                          Wr1_0, br1_0, Wr2_0, br2_0,
                          Wr1_1, br1_1, Wr2_1, br2_1)
    part0 = _layer0(src1, dst1, a0.reshape(EPAD), h0p)
    ws0v = jnp.broadcast_to(ws_0.reshape(1), (16,))
    h0a, part4 = _layer1(src1, dst1, a1.reshape(EPAD), b1r.reshape(EPAD),
                         rx, ry, rz, h0p, part0, ws0v)
    out4 = _combine(part4[0], part4[1], h0a, ws_1)
    return out4[:, :N].T
